# Initial kernel scaffold; baseline (speedup 1.0000x reference)
import jax
import jax.numpy as jnp
from jax.experimental import pallas as pl


def kernel(x, edge_index, edge_attr, edge_label_index, W_gat1, a_src1, a_dst1, b_gat1, W_gat2, a_src2, a_dst2, b_gat2, W_c1, b_c1, W_c2, b_c2, W_hop, b_hop):
    raise NotImplementedError("write your pallas kernel here")


# speedup vs baseline: 1.0836x; 1.0836x over previous
import jax
import jax.numpy as jnp
from jax.experimental import pallas as pl

SEQ = 4; N = 4096; E = 131072; D_IN = 128; HID = 64; OUT = 64; H1 = 4


def _gcn_conv(x, src, dst, ew, W, b, n):
    loop = jnp.arange(n, dtype=src.dtype)
    s = jnp.concatenate([src, loop])
    d = jnp.concatenate([dst, loop])
    w = jnp.concatenate([ew, jnp.ones((n,), x.dtype)])
    deg = jax.ops.segment_sum(w, d, num_segments=n)
    dinv = jnp.where(deg > 0, deg ** -0.5, 0.0)
    norm = dinv[s] * w * dinv[d]
    h = x @ W
    out = jax.ops.segment_sum(norm[:, None] * h[s], d, num_segments=n)
    return out + b


def _gat_conv(x, src, dst, W, a_s, a_d, b, n, heads, dh, concat):
    loop = jnp.arange(n, dtype=src.dtype)
    s = jnp.concatenate([src, loop])
    d = jnp.concatenate([dst, loop])
    h = (x @ W).reshape(n, heads, dh)
    es = jnp.sum(h * a_s[None], -1)
    ed = jnp.sum(h * a_d[None], -1)
    e = jax.nn.leaky_relu(es[s] + ed[d], 0.2)
    ex = jnp.exp(e)
    den = jax.ops.segment_sum(ex, d, num_segments=n)
    alpha = ex / (den[d] + 1e-16)
    out = jax.ops.segment_sum(alpha[:, :, None] * h[s], d, num_segments=n)
    out = out.reshape(n, heads * dh) if concat else out.mean(1)
    return out + b


def kernel(x, edge_index, edge_attr, edge_label_index, W_gat1, a_src1, a_dst1, b_gat1, W_gat2, a_src2, a_dst2, b_gat2, W_c1, b_c1, W_c2, b_c2, W_hop, b_hop):
    W_hop_f = W_hop[:HID] + W_hop[HID:]
    hop = []
    for t in range(SEQ):
        xt = x[t]
        s = edge_index[t, 0]
        d = edge_index[t, 1]
        ew = edge_attr[t]
        x_low = jax.nn.relu(_gcn_conv(xt, s, d, ew, W_c1, b_c1, N))
        x_low = jax.nn.relu(_gcn_conv(x_low, s, d, ew, W_c2, b_c2, N))
        hop.append(x_low @ W_hop_f + b_hop)
    hop_out = jnp.stack(hop, 0)
    ei = jnp.transpose(edge_index, (1, 0, 2)).reshape(2, -1)
    nt = SEQ * N
    xf = x.reshape(-1, D_IN)
    z = jax.nn.relu(_gat_conv(xf, ei[0], ei[1], W_gat1, a_src1, a_dst1, b_gat1, nt, H1, HID, True))
    z = _gat_conv(z, ei[0], ei[1], W_gat2, a_src2, a_dst2, b_gat2, nt, 1, OUT, False)
    link_pred = jnp.sum(z[edge_label_index[0]] * z[edge_label_index[1]], axis=-1)
    return (link_pred, hop_out)


# speedup vs baseline: 53.0828x; 48.9859x over previous
import functools

import jax
import jax.numpy as jnp
from jax import lax
from jax.experimental import pallas as pl
from jax.experimental.pallas import tpu as pltpu
from jax.experimental.pallas import tpu_sc as plsc

SEQ = 4; N = 4096; E = 131072; D_IN = 128; HID = 64; OUT = 64; H1 = 4
NT = SEQ * N
NE = SEQ * E
NC, NS, NW = 2, 16, 32
CH = 128
EPW = NE // NW
NCH = EPW // CH
LPW = E // NW
LCH = LPW // CH
FP = 128

_f32 = jnp.float32
_NOLAYOUT = pltpu.CompilerParams(needs_layout_passes=False)


def _mesh():
    return plsc.VectorSubcoreMesh(
        core_axis_name="c", subcore_axis_name="s",
        num_cores=NC, num_subcores=NS)


def _leaky_exp(x):
    return jnp.exp(jnp.maximum(x, 0.2 * x))


def _t1_body(x_ref, w_ref, degp_ref, hp_ref, dinv_ref):
    deg = degp_ref[0] + degp_ref[1] + 1.0
    dinv = lax.rsqrt(deg)
    h = jnp.dot(x_ref[...], w_ref[...], preferred_element_type=_f32)
    hp_ref[...] = h * dinv
    dinv_ref[...] = dinv


def _t2_body(agg_ref, hp_ref, dinv_ref, b_ref, w2_ref, hp2_ref):
    dinv = dinv_ref[...]
    z = jax.nn.relu(
        dinv * (agg_ref[:, :HID] + hp_ref[:, :HID]) + b_ref[...])
    hp2_ref[...] = jnp.dot(z, w2_ref[...], preferred_element_type=_f32) * dinv


def _t3_body(agg_ref, hp2_ref, dinv_ref, b2_ref, wh_ref, bh_ref, hop_ref):
    z = jax.nn.relu(
        dinv_ref[...] * (agg_ref[:, :HID] + hp2_ref[:, :HID]) + b2_ref[...])
    hop_ref[...] = (
        jnp.dot(z, wh_ref[...], preferred_element_type=_f32) + bh_ref[...])


def _t4_body(x_ref, wg_ref, a_ref, hg_ref, esd_ref):
    h = jnp.dot(x_ref[...], wg_ref[...], preferred_element_type=_f32)
    hg_ref[...] = h
    esd_ref[...] = jnp.dot(h, a_ref[...], preferred_element_type=_f32)


def _t5a_body(pn_ref, pd_ref, hg_ref, esd_ref, b_ref, r_ref, z_ref, *, nh, relu):
    wself = _leaky_exp(esd_ref[:, 0:nh] + esd_ref[:, nh:2 * nh])
    wfull = jnp.dot(wself, r_ref[...], preferred_element_type=_f32)
    num = pn_ref[0] + pn_ref[1] + wfull * hg_ref[...]
    den = jnp.dot(jnp.sum(pd_ref[...], axis=0) + wself, r_ref[...],
                  preferred_element_type=_f32) + 1e-16
    z = num / den + b_ref[...]
    z_ref[...] = jax.nn.relu(z) if relu else z


def _t5b_body(hg_ref, b_ref, z_ref, *, relu):
    z = hg_ref[...] + b_ref[...]
    z_ref[...] = jax.nn.relu(z) if relu else z


def _mm_epilogue_call(x, w, a, bm):
    m, k = x.shape
    n = w.shape[1]
    return pl.pallas_call(
        _t4_body,
        grid=(m // bm,),
        in_specs=[
            pl.BlockSpec((bm, k), lambda i: (i, 0)),
            pl.BlockSpec((k, n), lambda i: (0, 0)),
            pl.BlockSpec((n, 8), lambda i: (0, 0)),
        ],
        out_specs=[
            pl.BlockSpec((bm, n), lambda i: (i, 0)),
            pl.BlockSpec((bm, 8), lambda i: (i, 0)),
        ],
        out_shape=[
            jax.ShapeDtypeStruct((m, n), _f32),
            jax.ShapeDtypeStruct((m, 8), _f32),
        ],
    )(x, w, a)


def _sc_deg_body(dst_hbm, ew_hbm, out_hbm, dstv, ewv, tab, sp, idxb):
    cid = lax.axis_index("c")
    sid = lax.axis_index("s")
    wid = cid * NS + sid
    pltpu.sync_copy(dst_hbm.at[wid], dstv)
    pltpu.sync_copy(ew_hbm.at[wid], ewv)
    zero16 = jnp.zeros((16,), _f32)

    @pl.loop(0, 128)
    def _zero(i):
        for j in range(8):
            tab[i, pl.ds(j * 16, 16)] = zero16

    pltpu.sync_copy(tab.at[pl.ds(0, 8)], sp.at[pl.ds(sid * 8, 8)])
    for j in range(8):
        idxb[pl.ds(j * 16, 16)] = lax.iota(jnp.int32, 16) + j * 16
    plsc.subcore_barrier()

    @pl.loop(0, NCH)
    def _scatter(c):
        for g in range(CH // 16):
            d = dstv[c, pl.ds(g * 16, 16)]
            w = ewv[c, pl.ds(g * 16, 16)]
            row = lax.shift_right_logical(d, 7)
            col = lax.bitwise_and(d, 127)
            plsc.addupdate_scatter(tab, [row, col], w)

    plsc.subcore_barrier()
    pltpu.sync_copy(tab, sp.at[idxb], add=True)
    plsc.subcore_barrier()
    pltpu.sync_copy(sp.at[pl.ds(sid * 8, 8)],
                    out_hbm.at[cid].at[pl.ds(sid * 8, 8)])


def _sc_deg(dstg, ew):
    kfn = pl.kernel(
        _sc_deg_body,
        out_type=jax.ShapeDtypeStruct((NC, 128, 128), _f32),
        mesh=_mesh(),
        compiler_params=_NOLAYOUT,
        scratch_types=[
            pltpu.VMEM((NCH, CH), jnp.int32),
            pltpu.VMEM((NCH, CH), _f32),
            pltpu.VMEM((128, 128), _f32),
            pltpu.VMEM_SHARED((128, 128), _f32),
            pltpu.VMEM((128,), jnp.int32),
        ],
    )
    return kfn(dstg, ew)


def _sc_gcn_body(hp_hbm, src_hbm, dst_hbm, ew_hbm, z_hbm, out_hbm,
                 srcv, dstv, ewv, rows, acc):
    cid = lax.axis_index("c")
    sid = lax.axis_index("s")
    wid = cid * NS + sid
    pltpu.sync_copy(src_hbm.at[wid], srcv)
    pltpu.sync_copy(dst_hbm.at[wid], dstv)
    pltpu.sync_copy(ew_hbm.at[wid], ewv)
    spt = N // NS
    for q in range(2):
        pltpu.sync_copy(z_hbm.at[pl.ds(sid * spt, spt)],
                        acc.at[pl.ds(sid * spt, spt)])
        plsc.subcore_barrier()

        @pl.loop(q * (NCH // 2), (q + 1) * (NCH // 2))
        def _chunk(c):
            pltpu.sync_copy(hp_hbm.at[srcv.at[c]], rows)

            @pl.loop(0, CH // 16)
            def _scale(g):
                w16 = ewv[c, pl.ds(g * 16, 16)]
                for r in range(16):
                    w = w16[r]
                    for j in range(4):
                        sl = pl.ds(j * 16, 16)
                        rows[g * 16 + r, sl] = rows[g * 16 + r, sl] * w

            pltpu.sync_copy(rows, acc.at[dstv.at[c]], add=True)

        plsc.subcore_barrier()
        pltpu.sync_copy(acc.at[pl.ds(sid * spt, spt)],
                        out_hbm.at[pl.ds((cid * 2 + q) * N + sid * spt, spt)])


def _sc_gcn_agg(hp, srcg, dstl, ew, zeros_acc):
    kfn = pl.kernel(
        _sc_gcn_body,
        out_type=jax.ShapeDtypeStruct((NT, FP), _f32),
        mesh=_mesh(),
        compiler_params=_NOLAYOUT,
        scratch_types=[
            pltpu.VMEM((NCH, CH), jnp.int32),
            pltpu.VMEM((NCH, CH), jnp.int32),
            pltpu.VMEM((NCH, CH), _f32),
            pltpu.VMEM((CH, FP), _f32),
            pltpu.VMEM_SHARED((N, FP), _f32),
        ],
    )
    return kfn(hp, srcg, dstl, ew, zeros_acc)


def _sc_gat_body(hg_hbm, src_hbm, dst_hbm, es_hbm, ed_hbm, z_hbm,
                 out_hbm, outden_hbm,
                 srcv, dstv, esv, edv, rows, dtab, acc,
                 *, nh, f):
    dcols = 128 * nh
    dshift = 7 + {1: 0, 2: 1, 4: 2}[nh]
    cid = lax.axis_index("c")
    sid = lax.axis_index("s")
    wid = cid * NS + sid
    pltpu.sync_copy(es_hbm, esv)
    pltpu.sync_copy(ed_hbm, edv)
    zero16 = jnp.zeros((16,), _f32)

    @pl.loop(0, 32)
    def _zero(i):
        for j in range(dcols // 16):
            dtab[i, pl.ds(j * 16, 16)] = zero16

    rows_per_tile = N // NS
    pltpu.sync_copy(z_hbm.at[pl.ds(sid * rows_per_tile, rows_per_tile)],
                    acc.at[pl.ds(sid * rows_per_tile, rows_per_tile)])
    plsc.subcore_barrier()

    for p in range(2):
        pltpu.sync_copy(src_hbm.at[wid].at[pl.ds(p * (NCH // 2), NCH // 2)],
                        srcv)
        pltpu.sync_copy(dst_hbm.at[wid].at[pl.ds(p * (NCH // 2), NCH // 2)],
                        dstv)

        @pl.loop(0, NCH // 2)
        def _chunk(c):
            pltpu.sync_copy(hg_hbm.at[srcv.at[c]], rows)

            @pl.loop(0, CH // 16)
            def _group(g):
                s = srcv[c, pl.ds(g * 16, 16)]
                d = dstv[c, pl.ds(g * 16, 16)]
                ws = []
                for h in range(nh):
                    es_g = plsc.load_gather(esv, [s * nh + h])
                    ed_g = plsc.load_gather(edv, [d * nh + h])
                    w = _leaky_exp(es_g + ed_g)
                    ws.append(w)
                    di = d * nh + h
                    row = lax.shift_right_logical(di, dshift)
                    col = lax.bitwise_and(di, dcols - 1)
                    plsc.addupdate_scatter(dtab, [row, col], w)
                for r in range(16):
                    for h in range(nh):
                        w = ws[h][r]
                        for j in range(4):
                            sl = pl.ds(h * 64 + j * 16, 16)
                            rows[g * 16 + r, sl] = rows[g * 16 + r, sl] * w

            pltpu.sync_copy(rows, acc.at[dstv.at[c]], add=True)

    plsc.subcore_barrier()
    pltpu.sync_copy(acc.at[pl.ds(sid * rows_per_tile, rows_per_tile)],
                    out_hbm.at[cid].at[pl.ds(sid * rows_per_tile, rows_per_tile)])
    pltpu.sync_copy(dtab, outden_hbm.at[wid])


def _sc_gat_agg(hg, src, dst, es_flat, ed_flat, zeros_acc, nh):
    f = hg.shape[1]
    kfn = pl.kernel(
        functools.partial(_sc_gat_body, nh=nh, f=f),
        out_type=(
            jax.ShapeDtypeStruct((NC, N, f), _f32),
            jax.ShapeDtypeStruct((NW, 32, 128 * nh), _f32),
        ),
        mesh=_mesh(),
        compiler_params=_NOLAYOUT,
        scratch_types=[
            pltpu.VMEM((NCH // 2, CH), jnp.int32),
            pltpu.VMEM((NCH // 2, CH), jnp.int32),
            pltpu.VMEM((N * nh,), _f32),
            pltpu.VMEM((N * nh,), _f32),
            pltpu.VMEM((CH, f), _f32),
            pltpu.VMEM((32, 128 * nh), _f32),
            pltpu.VMEM_SHARED((N, f), _f32),
        ],
    )
    return kfn(hg, src, dst, es_flat, ed_flat, zeros_acc)


def _sc_link_body(z_hbm, a_hbm, b_hbm, out_hbm, av, bv, ra, rb, ob):
    cid = lax.axis_index("c")
    sid = lax.axis_index("s")
    wid = cid * NS + sid
    pltpu.sync_copy(a_hbm.at[wid], av)
    pltpu.sync_copy(b_hbm.at[wid], bv)
    lanes = lax.iota(jnp.int32, 16)

    @pl.loop(0, LCH)
    def _chunk(c):
        pltpu.sync_copy(z_hbm.at[av.at[c]], ra)
        pltpu.sync_copy(z_hbm.at[bv.at[c]], rb)

        @pl.loop(0, CH // 16)
        def _group(g):
            out = jnp.zeros((16,), _f32)
            for r in range(16):
                rr = g * 16 + r
                acc = ra[rr, pl.ds(0, 16)] * rb[rr, pl.ds(0, 16)]
                for j in range(1, 4):
                    acc = acc + ra[rr, pl.ds(j * 16, 16)] * rb[rr, pl.ds(j * 16, 16)]
                out = jnp.where(lanes == r, jnp.sum(acc), out)
            ob[pl.ds(g * 16, 16)] = out

        pltpu.sync_copy(ob, out_hbm.at[pl.ds(wid * LPW + c * CH, CH)])


def _sc_link(z, aidx, bidx):
    kfn = pl.kernel(
        _sc_link_body,
        out_type=jax.ShapeDtypeStruct((E,), _f32),
        mesh=_mesh(),
        compiler_params=_NOLAYOUT,
        scratch_types=[
            pltpu.VMEM((LCH, CH), jnp.int32),
            pltpu.VMEM((LCH, CH), jnp.int32),
            pltpu.VMEM((CH, FP), _f32),
            pltpu.VMEM((CH, FP), _f32),
            pltpu.VMEM((CH,), _f32),
        ],
    )
    return kfn(z, aidx, bidx)


def kernel(x, edge_index, edge_attr, edge_label_index, W_gat1, a_src1, a_dst1,
           b_gat1, W_gat2, a_src2, a_dst2, b_gat2, W_c1, b_c1, W_c2, b_c2,
           W_hop, b_hop):
    ei = edge_index.astype(jnp.int32)
    eli = edge_label_index.astype(jnp.int32)
    x2d = x.reshape(NT, D_IN)

    offs = (jnp.arange(SEQ, dtype=jnp.int32) * N)[:, None]
    src_raw = ei[:, 0, :].reshape(NW, NCH, CH)
    dst_raw = ei[:, 1, :].reshape(NW, NCH, CH)
    src_g_flat = ei[:, 0, :] + offs
    dst_g = (ei[:, 1, :] + offs).reshape(NW, NCH, CH)

    def _gcn_layout(a):
        b = a.reshape(2, 2, NS, NCH // 2, CH)
        return b.transpose(0, 2, 1, 3, 4).reshape(NW, NCH, CH)

    src_gcn = _gcn_layout(src_g_flat)
    dst_gcn = _gcn_layout(ei[:, 1, :])
    ew_gcn = _gcn_layout(edge_attr)
    ew = edge_attr.reshape(NW, NCH, CH)
    aidx = eli[0].reshape(NW, LCH, CH)
    bidx = eli[1].reshape(NW, LCH, CH)

    W_hop_f = W_hop[:HID] + W_hop[HID:]
    W_c1p = jnp.concatenate([W_c1, jnp.zeros((D_IN, FP - HID), _f32)], axis=1)
    W_c2p = jnp.concatenate([W_c2, jnp.zeros((HID, FP - HID), _f32)], axis=1)
    W_g2p = jnp.concatenate(
        [W_gat2, jnp.zeros((H1 * HID, FP - OUT), _f32)], axis=1)
    b1 = b_c1.reshape(1, HID)
    b2 = b_c2.reshape(1, HID)
    bh = b_hop.reshape(1, N)
    bg1 = b_gat1.reshape(1, H1 * HID)
    bg2 = jnp.concatenate([b_gat2, jnp.zeros((FP - OUT,), _f32)]).reshape(1, FP)
    eye4 = jnp.eye(H1, dtype=_f32)
    A1 = jnp.concatenate(
        [(eye4[:, None, :] * a_src1[:, :, None]).reshape(H1 * HID, H1),
         (eye4[:, None, :] * a_dst1[:, :, None]).reshape(H1 * HID, H1)], axis=1)
    A2 = jnp.concatenate(
        [a_src2.T, a_dst2.T, jnp.zeros((OUT, 6), _f32)], axis=1)
    A2p = jnp.concatenate([A2, jnp.zeros((FP - OUT, 8), _f32)], axis=0)
    R4 = jnp.repeat(eye4, HID, axis=1)
    R1 = jnp.concatenate(
        [jnp.ones((1, OUT), _f32), jnp.zeros((1, FP - OUT), _f32)], axis=1)

    zeros_acc = jnp.zeros((N, FP), _f32)

    degp = _sc_deg(dst_g, ew).reshape(NC, NT, 1)

    bm = 256
    hp1, dinv = pl.pallas_call(
        _t1_body,
        grid=(NT // bm,),
        in_specs=[
            pl.BlockSpec((bm, D_IN), lambda i: (i, 0)),
            pl.BlockSpec((D_IN, FP), lambda i: (0, 0)),
            pl.BlockSpec((NC, bm, 1), lambda i: (0, i, 0)),
        ],
        out_specs=[
            pl.BlockSpec((bm, FP), lambda i: (i, 0)),
            pl.BlockSpec((bm, 1), lambda i: (i, 0)),
        ],
        out_shape=[
            jax.ShapeDtypeStruct((NT, FP), _f32),
            jax.ShapeDtypeStruct((NT, 1), _f32),
        ],
    )(x2d, W_c1p, degp)

    agg1 = _sc_gcn_agg(hp1, src_gcn, dst_gcn, ew_gcn, zeros_acc)

    hp2 = pl.pallas_call(
        _t2_body,
        grid=(NT // bm,),
        in_specs=[
            pl.BlockSpec((bm, FP), lambda i: (i, 0)),
            pl.BlockSpec((bm, FP), lambda i: (i, 0)),
            pl.BlockSpec((bm, 1), lambda i: (i, 0)),
            pl.BlockSpec((1, HID), lambda i: (0, 0)),
            pl.BlockSpec((HID, FP), lambda i: (0, 0)),
        ],
        out_specs=pl.BlockSpec((bm, FP), lambda i: (i, 0)),
        out_shape=jax.ShapeDtypeStruct((NT, FP), _f32),
    )(agg1, hp1, dinv, b1, W_c2p)

    agg2 = _sc_gcn_agg(hp2, src_gcn, dst_gcn, ew_gcn, zeros_acc)

    bn = 512
    hop2d = pl.pallas_call(
        _t3_body,
        grid=(NT // bm, N // bn),
        in_specs=[
            pl.BlockSpec((bm, FP), lambda i, j: (i, 0)),
            pl.BlockSpec((bm, FP), lambda i, j: (i, 0)),
            pl.BlockSpec((bm, 1), lambda i, j: (i, 0)),
            pl.BlockSpec((1, HID), lambda i, j: (0, 0)),
            pl.BlockSpec((HID, bn), lambda i, j: (0, j)),
            pl.BlockSpec((1, bn), lambda i, j: (0, j)),
        ],
        out_specs=pl.BlockSpec((bm, bn), lambda i, j: (i, j)),
        out_shape=jax.ShapeDtypeStruct((NT, N), _f32),
    )(agg2, hp2, dinv, b2, W_hop_f, bh)
    hop_out = hop2d.reshape(SEQ, N, N)

    hg1, esd1 = _mm_epilogue_call(x2d, W_gat1, A1, bm)
    pn_halves, pd_halves = [], []
    for q in range(2):
        es_q = esd1[:N, 2 * q:2 * q + 2].reshape(-1)
        ed_q = esd1[:N, H1 + 2 * q:H1 + 2 * q + 2].reshape(-1)
        pn_q, pd_q = _sc_gat_agg(hg1[:, 128 * q:128 * (q + 1)], src_raw,
                                 dst_raw, es_q, ed_q, zeros_acc, 2)
        pn_halves.append(pn_q)
        pd_halves.append(pd_q.reshape(NW, N, 2))
    pn1 = jnp.concatenate(pn_halves, axis=2)
    pd1r = jnp.concatenate(pd_halves, axis=2)

    z1_low = pl.pallas_call(
        functools.partial(_t5a_body, nh=H1, relu=True),
        grid=(N // bm,),
        in_specs=[
            pl.BlockSpec((NC, bm, H1 * HID), lambda i: (0, i, 0)),
            pl.BlockSpec((NW, bm, H1), lambda i: (0, i, 0)),
            pl.BlockSpec((bm, H1 * HID), lambda i: (i, 0)),
            pl.BlockSpec((bm, 8), lambda i: (i, 0)),
            pl.BlockSpec((1, H1 * HID), lambda i: (0, 0)),
            pl.BlockSpec((H1, H1 * HID), lambda i: (0, 0)),
        ],
        out_specs=pl.BlockSpec((bm, H1 * HID), lambda i: (i, 0)),
        out_shape=jax.ShapeDtypeStruct((N, H1 * HID), _f32),
    )(pn1, pd1r, hg1, esd1, bg1, R4)

    bhigh = 512
    z1_high = pl.pallas_call(
        functools.partial(_t5b_body, relu=True),
        grid=((NT - N) // bhigh,),
        in_specs=[
            pl.BlockSpec((bhigh, H1 * HID), lambda i: (i + N // bhigh, 0)),
            pl.BlockSpec((1, H1 * HID), lambda i: (0, 0)),
        ],
        out_specs=pl.BlockSpec((bhigh, H1 * HID), lambda i: (i, 0)),
        out_shape=jax.ShapeDtypeStruct((NT - N, H1 * HID), _f32),
    )(hg1, bg1)
    z1 = jnp.concatenate([z1_low, z1_high], axis=0)

    hg2, esd2 = _mm_epilogue_call(z1, W_g2p, A2p, bm)
    es2_flat = esd2[:N, 0]
    ed2_flat = esd2[:N, 1]
    pn2, pd2 = _sc_gat_agg(hg2, src_raw, dst_raw, es2_flat, ed2_flat,
                           zeros_acc, 1)
    pd2r = pd2.reshape(NW, N, 1)

    z2_low = pl.pallas_call(
        functools.partial(_t5a_body, nh=1, relu=False),
        grid=(N // bm,),
        in_specs=[
            pl.BlockSpec((NC, bm, FP), lambda i: (0, i, 0)),
            pl.BlockSpec((NW, bm, 1), lambda i: (0, i, 0)),
            pl.BlockSpec((bm, FP), lambda i: (i, 0)),
            pl.BlockSpec((bm, 8), lambda i: (i, 0)),
            pl.BlockSpec((1, FP), lambda i: (0, 0)),
            pl.BlockSpec((1, FP), lambda i: (0, 0)),
        ],
        out_specs=pl.BlockSpec((bm, FP), lambda i: (i, 0)),
        out_shape=jax.ShapeDtypeStruct((N, FP), _f32),
    )(pn2, pd2r, hg2, esd2, bg2, R1)

    z2_high = pl.pallas_call(
        functools.partial(_t5b_body, relu=False),
        grid=((NT - N) // bhigh,),
        in_specs=[
            pl.BlockSpec((bhigh, FP), lambda i: (i + N // bhigh, 0)),
            pl.BlockSpec((1, FP), lambda i: (0, 0)),
        ],
        out_specs=pl.BlockSpec((bhigh, FP), lambda i: (i, 0)),
        out_shape=jax.ShapeDtypeStruct((NT - N, FP), _f32),
    )(hg2, bg2)
    z2 = jnp.concatenate([z2_low, z2_high], axis=0)

    link_pred = _sc_link(z2, aidx, bidx)
    return (link_pred, hop_out)


# speedup vs baseline: 73.8996x; 1.3922x over previous
import functools

import jax
import jax.numpy as jnp
from jax import lax
from jax.experimental import pallas as pl
from jax.experimental.pallas import tpu as pltpu
from jax.experimental.pallas import tpu_sc as plsc

SEQ = 4; N = 4096; E = 131072; D_IN = 128; HID = 64; OUT = 64; H1 = 4
NT = SEQ * N
NE = SEQ * E
NC, NS, NW = 2, 16, 32
CH = 128
EPW = NE // NW
NCH = EPW // CH
LPW = E // NW
LCH = LPW // CH
FP = 128

_f32 = jnp.float32
_NOLAYOUT = pltpu.CompilerParams(needs_layout_passes=False)


def _mesh():
    return plsc.VectorSubcoreMesh(
        core_axis_name="c", subcore_axis_name="s",
        num_cores=NC, num_subcores=NS)


def _leaky_exp(x):
    return jnp.exp(jnp.maximum(x, 0.2 * x))


def _t1_body(x_ref, w_ref, degp_ref, hp_ref, dinv_ref):
    deg = degp_ref[0] + degp_ref[1] + 1.0
    dinv = lax.rsqrt(deg)
    h = jnp.dot(x_ref[...], w_ref[...], preferred_element_type=_f32)
    hp_ref[...] = h * dinv
    dinv_ref[...] = dinv


def _t2_body(agg_ref, hp_ref, dinv_ref, b_ref, w2_ref, hp2_ref):
    dinv = dinv_ref[...]
    z = jax.nn.relu(
        dinv * (agg_ref[:, :HID] + hp_ref[:, :HID]) + b_ref[...])
    hp2_ref[...] = jnp.dot(z, w2_ref[...], preferred_element_type=_f32) * dinv


def _t3_body(agg_ref, hp2_ref, dinv_ref, b2_ref, wh_ref, bh_ref, hop_ref):
    z = jax.nn.relu(
        dinv_ref[...] * (agg_ref[:, :HID] + hp2_ref[:, :HID]) + b2_ref[...])
    hop_ref[...] = (
        jnp.dot(z, wh_ref[...], preferred_element_type=_f32) + bh_ref[...])


def _t4_body(x_ref, wg_ref, a_ref, hg_ref, esd_ref):
    h = jnp.dot(x_ref[...], wg_ref[...], preferred_element_type=_f32)
    hg_ref[...] = h
    esd_ref[...] = jnp.dot(h, a_ref[...], preferred_element_type=_f32)


def _t5a_body(pn_ref, pd_ref, hg_ref, esd_ref, b_ref, r_ref, z_ref, *, nh, relu):
    wself = _leaky_exp(esd_ref[:, 0:nh] + esd_ref[:, nh:2 * nh])
    wfull = jnp.dot(wself, r_ref[...], preferred_element_type=_f32)
    num = pn_ref[0] + pn_ref[1] + wfull * hg_ref[...]
    den = jnp.dot(jnp.sum(pd_ref[...], axis=0) + wself, r_ref[...],
                  preferred_element_type=_f32) + 1e-16
    z = num / den + b_ref[...]
    z_ref[...] = jax.nn.relu(z) if relu else z


def _t5b_body(hg_ref, b_ref, z_ref, *, relu):
    z = hg_ref[...] + b_ref[...]
    z_ref[...] = jax.nn.relu(z) if relu else z


def _mm_epilogue_call(x, w, a, bm):
    m, k = x.shape
    n = w.shape[1]
    return pl.pallas_call(
        _t4_body,
        grid=(m // bm,),
        in_specs=[
            pl.BlockSpec((bm, k), lambda i: (i, 0)),
            pl.BlockSpec((k, n), lambda i: (0, 0)),
            pl.BlockSpec((n, 8), lambda i: (0, 0)),
        ],
        out_specs=[
            pl.BlockSpec((bm, n), lambda i: (i, 0)),
            pl.BlockSpec((bm, 8), lambda i: (i, 0)),
        ],
        out_shape=[
            jax.ShapeDtypeStruct((m, n), _f32),
            jax.ShapeDtypeStruct((m, 8), _f32),
        ],
    )(x, w, a)


def _sc_deg_body(dst_hbm, ew_hbm, out_hbm, dstv, ewv, tab, sp, idxb):
    cid = lax.axis_index("c")
    sid = lax.axis_index("s")
    wid = cid * NS + sid
    pltpu.sync_copy(dst_hbm.at[wid], dstv)
    pltpu.sync_copy(ew_hbm.at[wid], ewv)
    zero16 = jnp.zeros((16,), _f32)

    @pl.loop(0, 128)
    def _zero(i):
        for j in range(8):
            tab[i, pl.ds(j * 16, 16)] = zero16

    pltpu.sync_copy(tab.at[pl.ds(0, 8)], sp.at[pl.ds(sid * 8, 8)])
    for j in range(8):
        idxb[pl.ds(j * 16, 16)] = lax.iota(jnp.int32, 16) + j * 16
    plsc.subcore_barrier()

    @pl.loop(0, NCH)
    def _scatter(c):
        for g in range(CH // 16):
            d = dstv[c, pl.ds(g * 16, 16)]
            w = ewv[c, pl.ds(g * 16, 16)]
            row = lax.shift_right_logical(d, 7)
            col = lax.bitwise_and(d, 127)
            plsc.addupdate_scatter(tab, [row, col], w)

    plsc.subcore_barrier()
    pltpu.sync_copy(tab, sp.at[idxb], add=True)
    plsc.subcore_barrier()
    pltpu.sync_copy(sp.at[pl.ds(sid * 8, 8)],
                    out_hbm.at[cid].at[pl.ds(sid * 8, 8)])


def _sc_deg(dstg, ew):
    kfn = pl.kernel(
        _sc_deg_body,
        out_type=jax.ShapeDtypeStruct((NC, 128, 128), _f32),
        mesh=_mesh(),
        compiler_params=_NOLAYOUT,
        scratch_types=[
            pltpu.VMEM((NCH, CH), jnp.int32),
            pltpu.VMEM((NCH, CH), _f32),
            pltpu.VMEM((128, 128), _f32),
            pltpu.VMEM_SHARED((128, 128), _f32),
            pltpu.VMEM((128,), jnp.int32),
        ],
    )
    return kfn(dstg, ew)


def _sc_gcn_body(hp_hbm, src_hbm, dst_hbm, ew_hbm, z_hbm, out_hbm,
                 srcv, dstv, ewv, rows0, rows1, acc, gsem0, gsem1):
    cid = lax.axis_index("c")
    sid = lax.axis_index("s")
    wid = cid * NS + sid
    pltpu.sync_copy(src_hbm.at[wid], srcv)
    pltpu.sync_copy(dst_hbm.at[wid], dstv)
    pltpu.sync_copy(ew_hbm.at[wid], ewv)
    rbufs = (rows0, rows1)
    gsems = (gsem0, gsem1)
    half = NCH // 2
    spt = N // NS
    for q in range(2):
        base = q * half
        pltpu.sync_copy(z_hbm.at[pl.ds(sid * spt, spt)],
                        acc.at[pl.ds(sid * spt, spt)])
        plsc.subcore_barrier()
        for b in range(2):
            pltpu.make_async_copy(
                hp_hbm.at[srcv.at[base + b]], rbufs[b], gsems[b]).start()

        @pl.loop(0, half // 2)
        def _pair(p):
            for b in range(2):
                c = base + 2 * p + b
                rows = rbufs[b]
                pltpu.make_async_copy(
                    hp_hbm.at[srcv.at[c]], rows, gsems[b]).wait()

                @pl.loop(0, CH // 16)
                def _scale(g):
                    w16 = ewv[c, pl.ds(g * 16, 16)]
                    for r in range(16):
                        w = w16[r]
                        for j in range(4):
                            sl = pl.ds(j * 16, 16)
                            rows[g * 16 + r, sl] = rows[g * 16 + r, sl] * w

                pltpu.sync_copy(rows, acc.at[dstv.at[c]], add=True)

                @pl.when(2 * p + b + 2 < half)
                def _prefetch():
                    pltpu.make_async_copy(
                        hp_hbm.at[srcv.at[c + 2]], rows, gsems[b]).start()

        plsc.subcore_barrier()
        pltpu.sync_copy(acc.at[pl.ds(sid * spt, spt)],
                        out_hbm.at[pl.ds((cid * 2 + q) * N + sid * spt, spt)])


def _sc_gcn_agg(hp, srcg, dstl, ew, zeros_acc):
    kfn = pl.kernel(
        _sc_gcn_body,
        out_type=jax.ShapeDtypeStruct((NT, FP), _f32),
        mesh=_mesh(),
        compiler_params=_NOLAYOUT,
        scratch_types=[
            pltpu.VMEM((NCH, CH), jnp.int32),
            pltpu.VMEM((NCH, CH), jnp.int32),
            pltpu.VMEM((NCH, CH), _f32),
            pltpu.VMEM((CH, FP), _f32),
            pltpu.VMEM((CH, FP), _f32),
            pltpu.VMEM_SHARED((N, FP), _f32),
            pltpu.SemaphoreType.DMA,
            pltpu.SemaphoreType.DMA,
        ],
    )
    return kfn(hp, srcg, dstl, ew, zeros_acc)


def _sc_gat_body(hg_hbm, src_hbm, dst_hbm, es_hbm, ed_hbm, z_hbm,
                 out_hbm, outden_hbm,
                 srcv, dstv, esv, edv, rows0, rows1, wbuf, dtab, acc,
                 gsem0, gsem1, *, nh, f):
    dcols = 128 * nh
    dshift = 7 + {1: 0, 2: 1, 4: 2}[nh]
    cid = lax.axis_index("c")
    sid = lax.axis_index("s")
    wid = cid * NS + sid
    pltpu.sync_copy(es_hbm, esv)
    pltpu.sync_copy(ed_hbm, edv)
    zero16 = jnp.zeros((16,), _f32)

    @pl.loop(0, 32)
    def _zero(i):
        for j in range(dcols // 16):
            dtab[i, pl.ds(j * 16, 16)] = zero16

    rows_per_tile = N // NS
    pltpu.sync_copy(z_hbm.at[pl.ds(sid * rows_per_tile, rows_per_tile)],
                    acc.at[pl.ds(sid * rows_per_tile, rows_per_tile)])
    plsc.subcore_barrier()

    rbufs = (rows0, rows1)
    gsems = (gsem0, gsem1)
    half = NCH // 2
    for p in range(2):
        pltpu.sync_copy(src_hbm.at[wid].at[pl.ds(p * half, half)], srcv)
        pltpu.sync_copy(dst_hbm.at[wid].at[pl.ds(p * half, half)], dstv)
        for b in range(2):
            pltpu.make_async_copy(
                hg_hbm.at[srcv.at[b]], rbufs[b], gsems[b]).start()

        @pl.loop(0, half // 2)
        def _pair(pp):
            for b in range(2):
                c = 2 * pp + b
                rows = rbufs[b]

                @pl.loop(0, CH // 16)
                def _att(g):
                    s = srcv[c, pl.ds(g * 16, 16)]
                    d = dstv[c, pl.ds(g * 16, 16)]
                    for h in range(nh):
                        es_g = plsc.load_gather(esv, [s * nh + h])
                        ed_g = plsc.load_gather(edv, [d * nh + h])
                        w = _leaky_exp(es_g + ed_g)
                        wbuf[h, pl.ds(g * 16, 16)] = w
                        di = d * nh + h
                        row = lax.shift_right_logical(di, dshift)
                        col = lax.bitwise_and(di, dcols - 1)
                        plsc.addupdate_scatter(dtab, [row, col], w)

                pltpu.make_async_copy(
                    hg_hbm.at[srcv.at[c]], rows, gsems[b]).wait()

                @pl.loop(0, CH // 16)
                def _scale(g):
                    for h in range(nh):
                        w16 = wbuf[h, pl.ds(g * 16, 16)]
                        for r in range(16):
                            w = w16[r]
                            for j in range(4):
                                sl = pl.ds(h * 64 + j * 16, 16)
                                rows[g * 16 + r, sl] = rows[g * 16 + r, sl] * w

                pltpu.sync_copy(rows, acc.at[dstv.at[c]], add=True)

                @pl.when(2 * pp + b + 2 < half)
                def _prefetch():
                    pltpu.make_async_copy(
                        hg_hbm.at[srcv.at[c + 2]], rows, gsems[b]).start()

    plsc.subcore_barrier()
    pltpu.sync_copy(acc.at[pl.ds(sid * rows_per_tile, rows_per_tile)],
                    out_hbm.at[cid].at[pl.ds(sid * rows_per_tile, rows_per_tile)])
    pltpu.sync_copy(dtab, outden_hbm.at[wid])


def _sc_gat_agg(hg, src, dst, es_flat, ed_flat, zeros_acc, nh):
    f = hg.shape[1]
    kfn = pl.kernel(
        functools.partial(_sc_gat_body, nh=nh, f=f),
        out_type=(
            jax.ShapeDtypeStruct((NC, N, f), _f32),
            jax.ShapeDtypeStruct((NW, 32, 128 * nh), _f32),
        ),
        mesh=_mesh(),
        compiler_params=_NOLAYOUT,
        scratch_types=[
            pltpu.VMEM((NCH // 2, CH), jnp.int32),
            pltpu.VMEM((NCH // 2, CH), jnp.int32),
            pltpu.VMEM((N * nh,), _f32),
            pltpu.VMEM((N * nh,), _f32),
            pltpu.VMEM((CH, f), _f32),
            pltpu.VMEM((CH, f), _f32),
            pltpu.VMEM((4, CH), _f32),
            pltpu.VMEM((32, 128 * nh), _f32),
            pltpu.VMEM_SHARED((N, f), _f32),
            pltpu.SemaphoreType.DMA,
            pltpu.SemaphoreType.DMA,
        ],
    )
    return kfn(hg, src, dst, es_flat, ed_flat, zeros_acc)


def _sc_link_body(z_hbm, a_hbm, b_hbm, out_hbm, av, bv, ra, rb, ob):
    cid = lax.axis_index("c")
    sid = lax.axis_index("s")
    wid = cid * NS + sid
    pltpu.sync_copy(a_hbm.at[wid], av)
    pltpu.sync_copy(b_hbm.at[wid], bv)
    lanes = lax.iota(jnp.int32, 16)

    @pl.loop(0, LCH)
    def _chunk(c):
        pltpu.sync_copy(z_hbm.at[av.at[c]], ra)
        pltpu.sync_copy(z_hbm.at[bv.at[c]], rb)

        @pl.loop(0, CH // 16)
        def _group(g):
            out = jnp.zeros((16,), _f32)
            for r in range(16):
                rr = g * 16 + r
                acc = ra[rr, pl.ds(0, 16)] * rb[rr, pl.ds(0, 16)]
                for j in range(1, 4):
                    acc = acc + ra[rr, pl.ds(j * 16, 16)] * rb[rr, pl.ds(j * 16, 16)]
                out = jnp.where(lanes == r, jnp.sum(acc), out)
            ob[pl.ds(g * 16, 16)] = out

        pltpu.sync_copy(ob, out_hbm.at[pl.ds(wid * LPW + c * CH, CH)])


def _sc_link(z, aidx, bidx):
    kfn = pl.kernel(
        _sc_link_body,
        out_type=jax.ShapeDtypeStruct((E,), _f32),
        mesh=_mesh(),
        compiler_params=_NOLAYOUT,
        scratch_types=[
            pltpu.VMEM((LCH, CH), jnp.int32),
            pltpu.VMEM((LCH, CH), jnp.int32),
            pltpu.VMEM((CH, FP), _f32),
            pltpu.VMEM((CH, FP), _f32),
            pltpu.VMEM((CH,), _f32),
        ],
    )
    return kfn(z, aidx, bidx)


def kernel(x, edge_index, edge_attr, edge_label_index, W_gat1, a_src1, a_dst1,
           b_gat1, W_gat2, a_src2, a_dst2, b_gat2, W_c1, b_c1, W_c2, b_c2,
           W_hop, b_hop):
    ei = edge_index.astype(jnp.int32)
    eli = edge_label_index.astype(jnp.int32)
    x2d = x.reshape(NT, D_IN)

    offs = (jnp.arange(SEQ, dtype=jnp.int32) * N)[:, None]
    src_raw = ei[:, 0, :].reshape(NW, NCH, CH)
    dst_raw = ei[:, 1, :].reshape(NW, NCH, CH)
    src_g_flat = ei[:, 0, :] + offs
    dst_g = (ei[:, 1, :] + offs).reshape(NW, NCH, CH)

    def _gcn_layout(a):
        b = a.reshape(2, 2, NS, NCH // 2, CH)
        return b.transpose(0, 2, 1, 3, 4).reshape(NW, NCH, CH)

    src_gcn = _gcn_layout(src_g_flat)
    dst_gcn = _gcn_layout(ei[:, 1, :])
    ew_gcn = _gcn_layout(edge_attr)
    ew = edge_attr.reshape(NW, NCH, CH)
    aidx = eli[0].reshape(NW, LCH, CH)
    bidx = eli[1].reshape(NW, LCH, CH)

    W_hop_f = W_hop[:HID] + W_hop[HID:]
    W_c1p = jnp.concatenate([W_c1, jnp.zeros((D_IN, FP - HID), _f32)], axis=1)
    W_c2p = jnp.concatenate([W_c2, jnp.zeros((HID, FP - HID), _f32)], axis=1)
    W_g2p = jnp.concatenate(
        [W_gat2, jnp.zeros((H1 * HID, FP - OUT), _f32)], axis=1)
    b1 = b_c1.reshape(1, HID)
    b2 = b_c2.reshape(1, HID)
    bh = b_hop.reshape(1, N)
    bg1 = b_gat1.reshape(1, H1 * HID)
    bg2 = jnp.concatenate([b_gat2, jnp.zeros((FP - OUT,), _f32)]).reshape(1, FP)
    eye4 = jnp.eye(H1, dtype=_f32)
    A1 = jnp.concatenate(
        [(eye4[:, None, :] * a_src1[:, :, None]).reshape(H1 * HID, H1),
         (eye4[:, None, :] * a_dst1[:, :, None]).reshape(H1 * HID, H1)], axis=1)
    A2 = jnp.concatenate(
        [a_src2.T, a_dst2.T, jnp.zeros((OUT, 6), _f32)], axis=1)
    A2p = jnp.concatenate([A2, jnp.zeros((FP - OUT, 8), _f32)], axis=0)
    R4 = jnp.repeat(eye4, HID, axis=1)
    R1 = jnp.concatenate(
        [jnp.ones((1, OUT), _f32), jnp.zeros((1, FP - OUT), _f32)], axis=1)

    zeros_acc = jnp.zeros((N, FP), _f32)

    degp = _sc_deg(dst_g, ew).reshape(NC, NT, 1)

    bm = 256
    hp1, dinv = pl.pallas_call(
        _t1_body,
        grid=(NT // bm,),
        in_specs=[
            pl.BlockSpec((bm, D_IN), lambda i: (i, 0)),
            pl.BlockSpec((D_IN, FP), lambda i: (0, 0)),
            pl.BlockSpec((NC, bm, 1), lambda i: (0, i, 0)),
        ],
        out_specs=[
            pl.BlockSpec((bm, FP), lambda i: (i, 0)),
            pl.BlockSpec((bm, 1), lambda i: (i, 0)),
        ],
        out_shape=[
            jax.ShapeDtypeStruct((NT, FP), _f32),
            jax.ShapeDtypeStruct((NT, 1), _f32),
        ],
    )(x2d, W_c1p, degp)

    agg1 = _sc_gcn_agg(hp1, src_gcn, dst_gcn, ew_gcn, zeros_acc)

    hp2 = pl.pallas_call(
        _t2_body,
        grid=(NT // bm,),
        in_specs=[
            pl.BlockSpec((bm, FP), lambda i: (i, 0)),
            pl.BlockSpec((bm, FP), lambda i: (i, 0)),
            pl.BlockSpec((bm, 1), lambda i: (i, 0)),
            pl.BlockSpec((1, HID), lambda i: (0, 0)),
            pl.BlockSpec((HID, FP), lambda i: (0, 0)),
        ],
        out_specs=pl.BlockSpec((bm, FP), lambda i: (i, 0)),
        out_shape=jax.ShapeDtypeStruct((NT, FP), _f32),
    )(agg1, hp1, dinv, b1, W_c2p)

    agg2 = _sc_gcn_agg(hp2, src_gcn, dst_gcn, ew_gcn, zeros_acc)

    bn = 512
    hop2d = pl.pallas_call(
        _t3_body,
        grid=(NT // bm, N // bn),
        in_specs=[
            pl.BlockSpec((bm, FP), lambda i, j: (i, 0)),
            pl.BlockSpec((bm, FP), lambda i, j: (i, 0)),
            pl.BlockSpec((bm, 1), lambda i, j: (i, 0)),
            pl.BlockSpec((1, HID), lambda i, j: (0, 0)),
            pl.BlockSpec((HID, bn), lambda i, j: (0, j)),
            pl.BlockSpec((1, bn), lambda i, j: (0, j)),
        ],
        out_specs=pl.BlockSpec((bm, bn), lambda i, j: (i, j)),
        out_shape=jax.ShapeDtypeStruct((NT, N), _f32),
    )(agg2, hp2, dinv, b2, W_hop_f, bh)
    hop_out = hop2d.reshape(SEQ, N, N)

    hg1, esd1 = _mm_epilogue_call(x2d, W_gat1, A1, bm)
    pn_halves, pd_halves = [], []
    for q in range(2):
        es_q = esd1[:N, 2 * q:2 * q + 2].reshape(-1)
        ed_q = esd1[:N, H1 + 2 * q:H1 + 2 * q + 2].reshape(-1)
        pn_q, pd_q = _sc_gat_agg(hg1[:, 128 * q:128 * (q + 1)], src_raw,
                                 dst_raw, es_q, ed_q, zeros_acc, 2)
        pn_halves.append(pn_q)
        pd_halves.append(pd_q.reshape(NW, N, 2))
    pn1 = jnp.concatenate(pn_halves, axis=2)
    pd1r = jnp.concatenate(pd_halves, axis=2)

    z1_low = pl.pallas_call(
        functools.partial(_t5a_body, nh=H1, relu=True),
        grid=(N // bm,),
        in_specs=[
            pl.BlockSpec((NC, bm, H1 * HID), lambda i: (0, i, 0)),
            pl.BlockSpec((NW, bm, H1), lambda i: (0, i, 0)),
            pl.BlockSpec((bm, H1 * HID), lambda i: (i, 0)),
            pl.BlockSpec((bm, 8), lambda i: (i, 0)),
            pl.BlockSpec((1, H1 * HID), lambda i: (0, 0)),
            pl.BlockSpec((H1, H1 * HID), lambda i: (0, 0)),
        ],
        out_specs=pl.BlockSpec((bm, H1 * HID), lambda i: (i, 0)),
        out_shape=jax.ShapeDtypeStruct((N, H1 * HID), _f32),
    )(pn1, pd1r, hg1, esd1, bg1, R4)

    bhigh = 512
    z1_high = pl.pallas_call(
        functools.partial(_t5b_body, relu=True),
        grid=((NT - N) // bhigh,),
        in_specs=[
            pl.BlockSpec((bhigh, H1 * HID), lambda i: (i + N // bhigh, 0)),
            pl.BlockSpec((1, H1 * HID), lambda i: (0, 0)),
        ],
        out_specs=pl.BlockSpec((bhigh, H1 * HID), lambda i: (i, 0)),
        out_shape=jax.ShapeDtypeStruct((NT - N, H1 * HID), _f32),
    )(hg1, bg1)
    z1 = jnp.concatenate([z1_low, z1_high], axis=0)

    hg2, esd2 = _mm_epilogue_call(z1, W_g2p, A2p, bm)
    es2_flat = esd2[:N, 0]
    ed2_flat = esd2[:N, 1]
    pn2, pd2 = _sc_gat_agg(hg2, src_raw, dst_raw, es2_flat, ed2_flat,
                           zeros_acc, 1)
    pd2r = pd2.reshape(NW, N, 1)

    z2_low = pl.pallas_call(
        functools.partial(_t5a_body, nh=1, relu=False),
        grid=(N // bm,),
        in_specs=[
            pl.BlockSpec((NC, bm, FP), lambda i: (0, i, 0)),
            pl.BlockSpec((NW, bm, 1), lambda i: (0, i, 0)),
            pl.BlockSpec((bm, FP), lambda i: (i, 0)),
            pl.BlockSpec((bm, 8), lambda i: (i, 0)),
            pl.BlockSpec((1, FP), lambda i: (0, 0)),
            pl.BlockSpec((1, FP), lambda i: (0, 0)),
        ],
        out_specs=pl.BlockSpec((bm, FP), lambda i: (i, 0)),
        out_shape=jax.ShapeDtypeStruct((N, FP), _f32),
    )(pn2, pd2r, hg2, esd2, bg2, R1)

    z2_high = pl.pallas_call(
        functools.partial(_t5b_body, relu=False),
        grid=((NT - N) // bhigh,),
        in_specs=[
            pl.BlockSpec((bhigh, FP), lambda i: (i + N // bhigh, 0)),
            pl.BlockSpec((1, FP), lambda i: (0, 0)),
        ],
        out_specs=pl.BlockSpec((bhigh, FP), lambda i: (i, 0)),
        out_shape=jax.ShapeDtypeStruct((NT - N, FP), _f32),
    )(hg2, bg2)
    z2 = jnp.concatenate([z2_low, z2_high], axis=0)

    link_pred = _sc_link(z2, aidx, bidx)
    return (link_pred, hop_out)


# speedup vs baseline: 86.0278x; 1.1641x over previous
import functools

import jax
import jax.numpy as jnp
from jax import lax
from jax.experimental import pallas as pl
from jax.experimental.pallas import tpu as pltpu
from jax.experimental.pallas import tpu_sc as plsc

SEQ = 4; N = 4096; E = 131072; D_IN = 128; HID = 64; OUT = 64; H1 = 4
NT = SEQ * N
NE = SEQ * E
NC, NS, NW = 2, 16, 32
CH = 128
EPW = NE // NW
NCH = EPW // CH
LPW = E // NW
LCH = LPW // CH
FP = 128

_f32 = jnp.float32
_NOLAYOUT = pltpu.CompilerParams(needs_layout_passes=False)


def _mesh():
    return plsc.VectorSubcoreMesh(
        core_axis_name="c", subcore_axis_name="s",
        num_cores=NC, num_subcores=NS)


def _leaky_exp(x):
    return jnp.exp(jnp.maximum(x, 0.2 * x))


def _t1_body(x_ref, w_ref, degp_ref, hp_ref, dinv_ref):
    deg = degp_ref[0] + degp_ref[1] + 1.0
    dinv = lax.rsqrt(deg)
    h = jnp.dot(x_ref[...], w_ref[...], preferred_element_type=_f32)
    hp_ref[...] = h * dinv
    dinv_ref[...] = dinv


def _t2_body(agg_ref, hp_ref, dinv_ref, b_ref, w2_ref, hp2_ref):
    dinv = dinv_ref[...]
    z = jax.nn.relu(
        dinv * (agg_ref[:, :HID] + hp_ref[:, :HID]) + b_ref[...])
    hp2_ref[...] = jnp.dot(z, w2_ref[...], preferred_element_type=_f32) * dinv


def _t3_body(agg_ref, hp2_ref, dinv_ref, b2_ref, wh_ref, bh_ref, hop_ref):
    z = jax.nn.relu(
        dinv_ref[...] * (agg_ref[:, :HID] + hp2_ref[:, :HID]) + b2_ref[...])
    hop_ref[...] = (
        jnp.dot(z, wh_ref[...], preferred_element_type=_f32) + bh_ref[...])


def _t4_body(x_ref, wg_ref, a_ref, hg_ref, esd_ref):
    h = jnp.dot(x_ref[...], wg_ref[...], preferred_element_type=_f32)
    hg_ref[...] = h
    esd_ref[...] = jnp.dot(h, a_ref[...], preferred_element_type=_f32)


def _t5a_body(pn_ref, pd_ref, hg_ref, esd_ref, b_ref, r_ref, z_ref, *, nh, relu):
    wself = _leaky_exp(esd_ref[:, 0:nh] + esd_ref[:, nh:2 * nh])
    wfull = jnp.dot(wself, r_ref[...], preferred_element_type=_f32)
    num = pn_ref[0] + pn_ref[1] + wfull * hg_ref[...]
    den = jnp.dot(jnp.sum(pd_ref[...], axis=0) + wself, r_ref[...],
                  preferred_element_type=_f32) + 1e-16
    z = num / den + b_ref[...]
    z_ref[...] = jax.nn.relu(z) if relu else z


def _t5b_body(hg_ref, b_ref, z_ref, *, relu):
    z = hg_ref[...] + b_ref[...]
    z_ref[...] = jax.nn.relu(z) if relu else z


def _mm_epilogue_call(x, w, a, bm):
    m, k = x.shape
    n = w.shape[1]
    return pl.pallas_call(
        _t4_body,
        grid=(m // bm,),
        in_specs=[
            pl.BlockSpec((bm, k), lambda i: (i, 0)),
            pl.BlockSpec((k, n), lambda i: (0, 0)),
            pl.BlockSpec((n, 8), lambda i: (0, 0)),
        ],
        out_specs=[
            pl.BlockSpec((bm, n), lambda i: (i, 0)),
            pl.BlockSpec((bm, 8), lambda i: (i, 0)),
        ],
        out_shape=[
            jax.ShapeDtypeStruct((m, n), _f32),
            jax.ShapeDtypeStruct((m, 8), _f32),
        ],
    )(x, w, a)


def _sc_deg_body(dst_hbm, ew_hbm, out_hbm, dstv, ewv, tab, sp, idxb):
    cid = lax.axis_index("c")
    sid = lax.axis_index("s")
    wid = cid * NS + sid
    pltpu.sync_copy(dst_hbm.at[wid], dstv)
    pltpu.sync_copy(ew_hbm.at[wid], ewv)
    zero16 = jnp.zeros((16,), _f32)

    @pl.loop(0, 128)
    def _zero(i):
        for j in range(8):
            tab[i, pl.ds(j * 16, 16)] = zero16

    pltpu.sync_copy(tab.at[pl.ds(0, 8)], sp.at[pl.ds(sid * 8, 8)])
    for j in range(8):
        idxb[pl.ds(j * 16, 16)] = lax.iota(jnp.int32, 16) + j * 16
    plsc.subcore_barrier()

    @pl.loop(0, NCH)
    def _scatter(c):
        for g in range(CH // 16):
            d = dstv[c, pl.ds(g * 16, 16)]
            w = ewv[c, pl.ds(g * 16, 16)]
            row = lax.shift_right_logical(d, 7)
            col = lax.bitwise_and(d, 127)
            plsc.addupdate_scatter(tab, [row, col], w)

    plsc.subcore_barrier()
    pltpu.sync_copy(tab, sp.at[idxb], add=True)
    plsc.subcore_barrier()
    pltpu.sync_copy(sp.at[pl.ds(sid * 8, 8)],
                    out_hbm.at[cid].at[pl.ds(sid * 8, 8)])


def _sc_deg(dstg, ew):
    kfn = pl.kernel(
        _sc_deg_body,
        out_type=jax.ShapeDtypeStruct((NC, 128, 128), _f32),
        mesh=_mesh(),
        compiler_params=_NOLAYOUT,
        scratch_types=[
            pltpu.VMEM((NCH, CH), jnp.int32),
            pltpu.VMEM((NCH, CH), _f32),
            pltpu.VMEM((128, 128), _f32),
            pltpu.VMEM_SHARED((128, 128), _f32),
            pltpu.VMEM((128,), jnp.int32),
        ],
    )
    return kfn(dstg, ew)


def _sc_gcn_body(hp_hbm, src_hbm, dst_hbm, ew_hbm, z_hbm, out_hbm,
                 srcv, dstv, ewv, rows0, rows1, acc, gsem0, gsem1):
    cid = lax.axis_index("c")
    sid = lax.axis_index("s")
    wid = cid * NS + sid
    pltpu.sync_copy(src_hbm.at[wid], srcv)
    pltpu.sync_copy(dst_hbm.at[wid], dstv)
    pltpu.sync_copy(ew_hbm.at[wid], ewv)
    rbufs = (rows0, rows1)
    gsems = (gsem0, gsem1)
    half = NCH // 2
    spt = N // NS
    for q in range(2):
        base = q * half
        pltpu.sync_copy(z_hbm.at[pl.ds(sid * spt, spt)],
                        acc.at[pl.ds(sid * spt, spt)])
        plsc.subcore_barrier()
        for b in range(2):
            pltpu.make_async_copy(
                hp_hbm.at[srcv.at[base + b]], rbufs[b], gsems[b]).start()

        @pl.loop(0, half // 2)
        def _pair(p):
            for b in range(2):
                c = base + 2 * p + b
                rows = rbufs[b]
                pltpu.make_async_copy(
                    hp_hbm.at[srcv.at[c]], rows, gsems[b]).wait()

                @pl.loop(0, CH // 16)
                def _scale(g):
                    w16 = ewv[c, pl.ds(g * 16, 16)]
                    for r in range(16):
                        w = w16[r]
                        for j in range(4):
                            sl = pl.ds(j * 16, 16)
                            rows[g * 16 + r, sl] = rows[g * 16 + r, sl] * w

                pltpu.sync_copy(rows, acc.at[dstv.at[c]], add=True)

                @pl.when(2 * p + b + 2 < half)
                def _prefetch():
                    pltpu.make_async_copy(
                        hp_hbm.at[srcv.at[c + 2]], rows, gsems[b]).start()

        plsc.subcore_barrier()
        pltpu.sync_copy(acc.at[pl.ds(sid * spt, spt)],
                        out_hbm.at[pl.ds((cid * 2 + q) * N + sid * spt, spt)])


def _sc_gcn_agg(hp, srcg, dstl, ew, zeros_acc):
    kfn = pl.kernel(
        _sc_gcn_body,
        out_type=jax.ShapeDtypeStruct((NT, FP), _f32),
        mesh=_mesh(),
        compiler_params=_NOLAYOUT,
        scratch_types=[
            pltpu.VMEM((NCH, CH), jnp.int32),
            pltpu.VMEM((NCH, CH), jnp.int32),
            pltpu.VMEM((NCH, CH), _f32),
            pltpu.VMEM((CH, FP), _f32),
            pltpu.VMEM((CH, FP), _f32),
            pltpu.VMEM_SHARED((N, FP), _f32),
            pltpu.SemaphoreType.DMA,
            pltpu.SemaphoreType.DMA,
        ],
    )
    return kfn(hp, srcg, dstl, ew, zeros_acc)


def _sc_gat_body(hg_hbm, src_hbm, dst_hbm, es_hbm, ed_hbm, z_hbm,
                 out_hbm, outden_hbm,
                 srcv, dstv, esv, edv, rows0, rows1, wbuf, dtab, acc,
                 gsem0, gsem1, *, nh, f):
    dcols = 128 * nh
    dshift = 7 + {1: 0, 2: 1, 4: 2}[nh]
    cid = lax.axis_index("c")
    sid = lax.axis_index("s")
    wid = cid * NS + sid
    pltpu.sync_copy(es_hbm, esv)
    pltpu.sync_copy(ed_hbm, edv)
    zero16 = jnp.zeros((16,), _f32)

    @pl.loop(0, 32)
    def _zero(i):
        for j in range(dcols // 16):
            dtab[i, pl.ds(j * 16, 16)] = zero16

    rows_per_tile = N // NS
    pltpu.sync_copy(z_hbm.at[pl.ds(sid * rows_per_tile, rows_per_tile)],
                    acc.at[pl.ds(sid * rows_per_tile, rows_per_tile)])
    plsc.subcore_barrier()

    rbufs = (rows0, rows1)
    gsems = (gsem0, gsem1)
    half = NCH // 2
    for p in range(2):
        pltpu.sync_copy(src_hbm.at[wid].at[pl.ds(p * half, half)], srcv)
        pltpu.sync_copy(dst_hbm.at[wid].at[pl.ds(p * half, half)], dstv)
        for b in range(2):
            pltpu.make_async_copy(
                hg_hbm.at[srcv.at[b]], rbufs[b], gsems[b]).start()

        @pl.loop(0, half // 2)
        def _pair(pp):
            for b in range(2):
                c = 2 * pp + b
                rows = rbufs[b]

                @pl.loop(0, CH // 16)
                def _att(g):
                    s = srcv[c, pl.ds(g * 16, 16)]
                    d = dstv[c, pl.ds(g * 16, 16)]
                    for h in range(nh):
                        es_g = plsc.load_gather(esv, [s * nh + h])
                        ed_g = plsc.load_gather(edv, [d * nh + h])
                        w = _leaky_exp(es_g + ed_g)
                        wbuf[h, pl.ds(g * 16, 16)] = w
                        di = d * nh + h
                        row = lax.shift_right_logical(di, dshift)
                        col = lax.bitwise_and(di, dcols - 1)
                        plsc.addupdate_scatter(dtab, [row, col], w)

                pltpu.make_async_copy(
                    hg_hbm.at[srcv.at[c]], rows, gsems[b]).wait()

                @pl.loop(0, CH // 16)
                def _scale(g):
                    for h in range(nh):
                        w16 = wbuf[h, pl.ds(g * 16, 16)]
                        for r in range(16):
                            w = w16[r]
                            for j in range(4):
                                sl = pl.ds(h * 64 + j * 16, 16)
                                rows[g * 16 + r, sl] = rows[g * 16 + r, sl] * w

                pltpu.sync_copy(rows, acc.at[dstv.at[c]], add=True)

                @pl.when(2 * pp + b + 2 < half)
                def _prefetch():
                    pltpu.make_async_copy(
                        hg_hbm.at[srcv.at[c + 2]], rows, gsems[b]).start()

    plsc.subcore_barrier()
    pltpu.sync_copy(acc.at[pl.ds(sid * rows_per_tile, rows_per_tile)],
                    out_hbm.at[cid].at[pl.ds(sid * rows_per_tile, rows_per_tile)])
    pltpu.sync_copy(dtab, outden_hbm.at[wid])


def _sc_gat_agg(hg, src, dst, es_flat, ed_flat, zeros_acc, nh):
    f = hg.shape[1]
    kfn = pl.kernel(
        functools.partial(_sc_gat_body, nh=nh, f=f),
        out_type=(
            jax.ShapeDtypeStruct((NC, N, f), _f32),
            jax.ShapeDtypeStruct((NW, 32, 128 * nh), _f32),
        ),
        mesh=_mesh(),
        compiler_params=_NOLAYOUT,
        scratch_types=[
            pltpu.VMEM((NCH // 2, CH), jnp.int32),
            pltpu.VMEM((NCH // 2, CH), jnp.int32),
            pltpu.VMEM((N * nh,), _f32),
            pltpu.VMEM((N * nh,), _f32),
            pltpu.VMEM((CH, f), _f32),
            pltpu.VMEM((CH, f), _f32),
            pltpu.VMEM((4, CH), _f32),
            pltpu.VMEM((32, 128 * nh), _f32),
            pltpu.VMEM_SHARED((N, f), _f32),
            pltpu.SemaphoreType.DMA,
            pltpu.SemaphoreType.DMA,
        ],
    )
    return kfn(hg, src, dst, es_flat, ed_flat, zeros_acc)


def _sc_link_body(z_hbm, a_hbm, b_hbm, out_hbm, av, bv, ra, rb, ob):
    cid = lax.axis_index("c")
    sid = lax.axis_index("s")
    wid = cid * NS + sid
    pltpu.sync_copy(a_hbm.at[wid], av)
    pltpu.sync_copy(b_hbm.at[wid], bv)
    lanes = lax.iota(jnp.int32, 16)

    @pl.loop(0, LCH)
    def _chunk(c):
        pltpu.sync_copy(z_hbm.at[av.at[c]], ra)
        pltpu.sync_copy(z_hbm.at[bv.at[c]], rb)

        @pl.loop(0, CH // 16)
        def _group(g):
            out = jnp.zeros((16,), _f32)
            for r in range(16):
                rr = g * 16 + r
                acc = ra[rr, pl.ds(0, 16)] * rb[rr, pl.ds(0, 16)]
                for j in range(1, 4):
                    acc = acc + ra[rr, pl.ds(j * 16, 16)] * rb[rr, pl.ds(j * 16, 16)]
                out = jnp.where(lanes == r, jnp.sum(acc), out)
            ob[pl.ds(g * 16, 16)] = out

        pltpu.sync_copy(ob, out_hbm.at[pl.ds(wid * LPW + c * CH, CH)])


def _sc_link(z, aidx, bidx):
    kfn = pl.kernel(
        _sc_link_body,
        out_type=jax.ShapeDtypeStruct((E,), _f32),
        mesh=_mesh(),
        compiler_params=_NOLAYOUT,
        scratch_types=[
            pltpu.VMEM((LCH, CH), jnp.int32),
            pltpu.VMEM((LCH, CH), jnp.int32),
            pltpu.VMEM((CH, FP), _f32),
            pltpu.VMEM((CH, FP), _f32),
            pltpu.VMEM((CH,), _f32),
        ],
    )
    return kfn(z, aidx, bidx)


def kernel(x, edge_index, edge_attr, edge_label_index, W_gat1, a_src1, a_dst1,
           b_gat1, W_gat2, a_src2, a_dst2, b_gat2, W_c1, b_c1, W_c2, b_c2,
           W_hop, b_hop):
    ei = edge_index.astype(jnp.int32)
    eli = edge_label_index.astype(jnp.int32)
    x2d = x.reshape(NT, D_IN)

    offs = (jnp.arange(SEQ, dtype=jnp.int32) * N)[:, None]
    src_raw = ei[:, 0, :].reshape(NW, NCH, CH)
    dst_raw = ei[:, 1, :].reshape(NW, NCH, CH)
    src_g_flat = ei[:, 0, :] + offs
    dst_g = (ei[:, 1, :] + offs).reshape(NW, NCH, CH)

    def _gcn_layout(a):
        b = a.reshape(2, 2, NS, NCH // 2, CH)
        return b.transpose(0, 2, 1, 3, 4).reshape(NW, NCH, CH)

    src_gcn = _gcn_layout(src_g_flat)
    dst_gcn = _gcn_layout(ei[:, 1, :])
    ew_gcn = _gcn_layout(edge_attr)
    ew = edge_attr.reshape(NW, NCH, CH)
    aidx = eli[0].reshape(NW, LCH, CH)
    bidx = eli[1].reshape(NW, LCH, CH)

    W_hop_f = W_hop[:HID] + W_hop[HID:]
    W_c1p = jnp.concatenate([W_c1, jnp.zeros((D_IN, FP - HID), _f32)], axis=1)
    W_c2p = jnp.concatenate([W_c2, jnp.zeros((HID, FP - HID), _f32)], axis=1)
    W_g2p = jnp.concatenate(
        [W_gat2, jnp.zeros((H1 * HID, FP - OUT), _f32)], axis=1)
    b1 = b_c1.reshape(1, HID)
    b2 = b_c2.reshape(1, HID)
    bh = b_hop.reshape(1, N)
    bg1 = b_gat1.reshape(1, H1 * HID)
    bg2 = jnp.concatenate([b_gat2, jnp.zeros((FP - OUT,), _f32)]).reshape(1, FP)
    eye4 = jnp.eye(H1, dtype=_f32)
    A1 = jnp.concatenate(
        [(eye4[:, None, :] * a_src1[:, :, None]).reshape(H1 * HID, H1),
         (eye4[:, None, :] * a_dst1[:, :, None]).reshape(H1 * HID, H1)], axis=1)
    A2 = jnp.concatenate(
        [a_src2.T, a_dst2.T, jnp.zeros((OUT, 6), _f32)], axis=1)
    A2p = jnp.concatenate([A2, jnp.zeros((FP - OUT, 8), _f32)], axis=0)
    R4 = jnp.repeat(eye4, HID, axis=1)
    R1 = jnp.concatenate(
        [jnp.ones((1, OUT), _f32), jnp.zeros((1, FP - OUT), _f32)], axis=1)

    zeros_acc = jnp.zeros((N, FP), _f32)

    degp = _sc_deg(dst_g, ew).reshape(NC, NT, 1)

    bm = 256
    hp1, dinv = pl.pallas_call(
        _t1_body,
        grid=(NT // bm,),
        in_specs=[
            pl.BlockSpec((bm, D_IN), lambda i: (i, 0)),
            pl.BlockSpec((D_IN, FP), lambda i: (0, 0)),
            pl.BlockSpec((NC, bm, 1), lambda i: (0, i, 0)),
        ],
        out_specs=[
            pl.BlockSpec((bm, FP), lambda i: (i, 0)),
            pl.BlockSpec((bm, 1), lambda i: (i, 0)),
        ],
        out_shape=[
            jax.ShapeDtypeStruct((NT, FP), _f32),
            jax.ShapeDtypeStruct((NT, 1), _f32),
        ],
    )(x2d, W_c1p, degp)

    agg1 = _sc_gcn_agg(hp1, src_gcn, dst_gcn, ew_gcn, zeros_acc)

    hp2 = pl.pallas_call(
        _t2_body,
        grid=(NT // bm,),
        in_specs=[
            pl.BlockSpec((bm, FP), lambda i: (i, 0)),
            pl.BlockSpec((bm, FP), lambda i: (i, 0)),
            pl.BlockSpec((bm, 1), lambda i: (i, 0)),
            pl.BlockSpec((1, HID), lambda i: (0, 0)),
            pl.BlockSpec((HID, FP), lambda i: (0, 0)),
        ],
        out_specs=pl.BlockSpec((bm, FP), lambda i: (i, 0)),
        out_shape=jax.ShapeDtypeStruct((NT, FP), _f32),
    )(agg1, hp1, dinv, b1, W_c2p)

    agg2 = _sc_gcn_agg(hp2, src_gcn, dst_gcn, ew_gcn, zeros_acc)

    hg1, esd1 = _mm_epilogue_call(x2d, W_gat1, A1, bm)
    pn_halves, pd_halves = [], []
    for q in range(2):
        es_q = esd1[:N, 2 * q:2 * q + 2].reshape(-1)
        ed_q = esd1[:N, H1 + 2 * q:H1 + 2 * q + 2].reshape(-1)
        pn_q, pd_q = _sc_gat_agg(hg1[:, 128 * q:128 * (q + 1)], src_raw,
                                 dst_raw, es_q, ed_q, zeros_acc, 2)
        pn_halves.append(pn_q)
        pd_halves.append(pd_q.reshape(NW, N, 2))
    pn1 = jnp.concatenate(pn_halves, axis=2)
    pd1r = jnp.concatenate(pd_halves, axis=2)

    z1_low = pl.pallas_call(
        functools.partial(_t5a_body, nh=H1, relu=True),
        grid=(N // bm,),
        in_specs=[
            pl.BlockSpec((NC, bm, H1 * HID), lambda i: (0, i, 0)),
            pl.BlockSpec((NW, bm, H1), lambda i: (0, i, 0)),
            pl.BlockSpec((bm, H1 * HID), lambda i: (i, 0)),
            pl.BlockSpec((bm, 8), lambda i: (i, 0)),
            pl.BlockSpec((1, H1 * HID), lambda i: (0, 0)),
            pl.BlockSpec((H1, H1 * HID), lambda i: (0, 0)),
        ],
        out_specs=pl.BlockSpec((bm, H1 * HID), lambda i: (i, 0)),
        out_shape=jax.ShapeDtypeStruct((N, H1 * HID), _f32),
    )(pn1, pd1r, hg1, esd1, bg1, R4)

    bhigh = 512
    z1_high = pl.pallas_call(
        functools.partial(_t5b_body, relu=True),
        grid=((NT - N) // bhigh,),
        in_specs=[
            pl.BlockSpec((bhigh, H1 * HID), lambda i: (i + N // bhigh, 0)),
            pl.BlockSpec((1, H1 * HID), lambda i: (0, 0)),
        ],
        out_specs=pl.BlockSpec((bhigh, H1 * HID), lambda i: (i, 0)),
        out_shape=jax.ShapeDtypeStruct((NT - N, H1 * HID), _f32),
    )(hg1, bg1)
    z1 = jnp.concatenate([z1_low, z1_high], axis=0)

    hg2, esd2 = _mm_epilogue_call(z1, W_g2p, A2p, bm)
    es2_flat = esd2[:N, 0]
    ed2_flat = esd2[:N, 1]
    pn2, pd2 = _sc_gat_agg(hg2, src_raw, dst_raw, es2_flat, ed2_flat,
                           zeros_acc, 1)
    pd2r = pd2.reshape(NW, N, 1)

    z2_low = pl.pallas_call(
        functools.partial(_t5a_body, nh=1, relu=False),
        grid=(N // bm,),
        in_specs=[
            pl.BlockSpec((NC, bm, FP), lambda i: (0, i, 0)),
            pl.BlockSpec((NW, bm, 1), lambda i: (0, i, 0)),
            pl.BlockSpec((bm, FP), lambda i: (i, 0)),
            pl.BlockSpec((bm, 8), lambda i: (i, 0)),
            pl.BlockSpec((1, FP), lambda i: (0, 0)),
            pl.BlockSpec((1, FP), lambda i: (0, 0)),
        ],
        out_specs=pl.BlockSpec((bm, FP), lambda i: (i, 0)),
        out_shape=jax.ShapeDtypeStruct((N, FP), _f32),
    )(pn2, pd2r, hg2, esd2, bg2, R1)

    z2_high = pl.pallas_call(
        functools.partial(_t5b_body, relu=False),
        grid=((NT - N) // bhigh,),
        in_specs=[
            pl.BlockSpec((bhigh, FP), lambda i: (i + N // bhigh, 0)),
            pl.BlockSpec((1, FP), lambda i: (0, 0)),
        ],
        out_specs=pl.BlockSpec((bhigh, FP), lambda i: (i, 0)),
        out_shape=jax.ShapeDtypeStruct((NT - N, FP), _f32),
    )(hg2, bg2)
    z2 = jnp.concatenate([z2_low, z2_high], axis=0)

    link_pred = _sc_link(z2, aidx, bidx)

    bmh = 512
    hop2d = pl.pallas_call(
        _t3_body,
        grid=(NT // bmh,),
        in_specs=[
            pl.BlockSpec((bmh, FP), lambda i: (i, 0)),
            pl.BlockSpec((bmh, FP), lambda i: (i, 0)),
            pl.BlockSpec((bmh, 1), lambda i: (i, 0)),
            pl.BlockSpec((1, HID), lambda i: (0, 0)),
            pl.BlockSpec((HID, N), lambda i: (0, 0)),
            pl.BlockSpec((1, N), lambda i: (0, 0)),
        ],
        out_specs=pl.BlockSpec((bmh, N), lambda i: (i, 0)),
        out_shape=jax.ShapeDtypeStruct((NT, N), _f32),
    )(agg2, hp2, dinv, b2, W_hop_f, bh)
    hop_out = hop2d.reshape(SEQ, N, N)
    return (link_pred, hop_out)


# speedup vs baseline: 86.4486x; 1.0049x over previous
import functools

import jax
import jax.numpy as jnp
from jax import lax
from jax.experimental import pallas as pl
from jax.experimental.pallas import tpu as pltpu
from jax.experimental.pallas import tpu_sc as plsc

SEQ = 4; N = 4096; E = 131072; D_IN = 128; HID = 64; OUT = 64; H1 = 4
NT = SEQ * N
NE = SEQ * E
NC, NS, NW = 2, 16, 32
CH = 128
EPW = NE // NW
NCH = EPW // CH
LPW = E // NW
LCH = LPW // CH
FP = 128

_f32 = jnp.float32
_NOLAYOUT = pltpu.CompilerParams(needs_layout_passes=False)


def _mesh():
    return plsc.VectorSubcoreMesh(
        core_axis_name="c", subcore_axis_name="s",
        num_cores=NC, num_subcores=NS)


def _leaky_exp(x):
    return jnp.exp(jnp.maximum(x, 0.2 * x))


def _t1_body(x_ref, w_ref, degp_ref, hp_ref, dinv_ref):
    deg = degp_ref[0] + degp_ref[1] + 1.0
    dinv = lax.rsqrt(deg)
    h = jnp.dot(x_ref[...], w_ref[...], preferred_element_type=_f32)
    hp_ref[...] = h * dinv
    dinv_ref[...] = dinv


def _t2_body(agg_ref, hp_ref, dinv_ref, b_ref, w2_ref, hp2_ref):
    dinv = dinv_ref[...]
    z = jax.nn.relu(
        dinv * (agg_ref[:, :HID] + hp_ref[:, :HID]) + b_ref[...])
    hp2_ref[...] = jnp.dot(z, w2_ref[...], preferred_element_type=_f32) * dinv


def _t3_body(agg_ref, hp2_ref, dinv_ref, b2_ref, wh_ref, bh_ref, hop_ref):
    z = jax.nn.relu(
        dinv_ref[...] * (agg_ref[:, :HID] + hp2_ref[:, :HID]) + b2_ref[...])
    hop_ref[...] = (
        jnp.dot(z, wh_ref[...], preferred_element_type=_f32) + bh_ref[...])


def _t4_body(x_ref, wg_ref, a_ref, hg_ref, esd_ref):
    h = jnp.dot(x_ref[...], wg_ref[...], preferred_element_type=_f32)
    hg_ref[...] = h
    esd_ref[...] = jnp.dot(h, a_ref[...], preferred_element_type=_f32)


def _t5a_body(pn_ref, pd_ref, hg_ref, esd_ref, b_ref, r_ref, z_ref, *, nh, relu):
    wself = _leaky_exp(esd_ref[:, 0:nh] + esd_ref[:, nh:2 * nh])
    wfull = jnp.dot(wself, r_ref[...], preferred_element_type=_f32)
    num = pn_ref[0] + pn_ref[1] + wfull * hg_ref[...]
    den = jnp.dot(jnp.sum(pd_ref[...], axis=0) + wself, r_ref[...],
                  preferred_element_type=_f32) + 1e-16
    z = num / den + b_ref[...]
    z_ref[...] = jax.nn.relu(z) if relu else z


def _t5b_body(hg_ref, b_ref, z_ref, *, relu):
    z = hg_ref[...] + b_ref[...]
    z_ref[...] = jax.nn.relu(z) if relu else z


def _mm_epilogue_call(x, w, a, bm):
    m, k = x.shape
    n = w.shape[1]
    return pl.pallas_call(
        _t4_body,
        grid=(m // bm,),
        in_specs=[
            pl.BlockSpec((bm, k), lambda i: (i, 0)),
            pl.BlockSpec((k, n), lambda i: (0, 0)),
            pl.BlockSpec((n, 8), lambda i: (0, 0)),
        ],
        out_specs=[
            pl.BlockSpec((bm, n), lambda i: (i, 0)),
            pl.BlockSpec((bm, 8), lambda i: (i, 0)),
        ],
        out_shape=[
            jax.ShapeDtypeStruct((m, n), _f32),
            jax.ShapeDtypeStruct((m, 8), _f32),
        ],
    )(x, w, a)


def _sc_deg_body(dst_hbm, ew_hbm, out_hbm, dstv, ewv, tab, sp, idxb):
    cid = lax.axis_index("c")
    sid = lax.axis_index("s")
    wid = cid * NS + sid
    pltpu.sync_copy(dst_hbm.at[wid], dstv)
    pltpu.sync_copy(ew_hbm.at[wid], ewv)
    zero16 = jnp.zeros((16,), _f32)

    @pl.loop(0, 128)
    def _zero(i):
        for j in range(8):
            tab[i, pl.ds(j * 16, 16)] = zero16

    pltpu.sync_copy(tab.at[pl.ds(0, 8)], sp.at[pl.ds(sid * 8, 8)])
    for j in range(8):
        idxb[pl.ds(j * 16, 16)] = lax.iota(jnp.int32, 16) + j * 16
    plsc.subcore_barrier()

    @pl.loop(0, NCH)
    def _scatter(c):
        for g in range(CH // 16):
            d = dstv[c, pl.ds(g * 16, 16)]
            w = ewv[c, pl.ds(g * 16, 16)]
            row = lax.shift_right_logical(d, 7)
            col = lax.bitwise_and(d, 127)
            plsc.addupdate_scatter(tab, [row, col], w)

    plsc.subcore_barrier()
    pltpu.sync_copy(tab, sp.at[idxb], add=True)
    plsc.subcore_barrier()
    pltpu.sync_copy(sp.at[pl.ds(sid * 8, 8)],
                    out_hbm.at[cid].at[pl.ds(sid * 8, 8)])


def _sc_deg(dstg, ew):
    kfn = pl.kernel(
        _sc_deg_body,
        out_type=jax.ShapeDtypeStruct((NC, 128, 128), _f32),
        mesh=_mesh(),
        compiler_params=_NOLAYOUT,
        scratch_types=[
            pltpu.VMEM((NCH, CH), jnp.int32),
            pltpu.VMEM((NCH, CH), _f32),
            pltpu.VMEM((128, 128), _f32),
            pltpu.VMEM_SHARED((128, 128), _f32),
            pltpu.VMEM((128,), jnp.int32),
        ],
    )
    return kfn(dstg, ew)


def _sc_gcn_body(hp_hbm, src_hbm, dst_hbm, ew_hbm, z_hbm, out_hbm,
                 srcv, dstv, ewv, rows0, rows1, rows2, rows3, acc,
                 gsem0, gsem1, gsem2, gsem3, ssem0, ssem1, ssem2, ssem3):
    cid = lax.axis_index("c")
    sid = lax.axis_index("s")
    wid = cid * NS + sid
    rbufs = (rows0, rows1, rows2, rows3)
    gsems = (gsem0, gsem1, gsem2, gsem3)
    ssems = (ssem0, ssem1, ssem2, ssem3)
    half = NCH // 2
    spt = N // NS
    for q in range(2):
        pltpu.sync_copy(src_hbm.at[wid].at[pl.ds(q * half, half)], srcv)
        pltpu.sync_copy(dst_hbm.at[wid].at[pl.ds(q * half, half)], dstv)
        pltpu.sync_copy(ew_hbm.at[wid].at[pl.ds(q * half, half)], ewv)
        pltpu.sync_copy(z_hbm.at[pl.ds(sid * spt, spt)],
                        acc.at[pl.ds(sid * spt, spt)])
        plsc.subcore_barrier()
        for b in range(2):
            pltpu.make_async_copy(
                hp_hbm.at[srcv.at[b]], rbufs[b], gsems[b]).start()

        @pl.loop(0, half // 4)
        def _quad(qq):
            for k in range(4):
                c = 4 * qq + k
                rows = rbufs[k]
                pltpu.make_async_copy(
                    hp_hbm.at[srcv.at[c]], rows, gsems[k]).wait()

                @pl.loop(0, CH // 16)
                def _scale(g):
                    w16 = ewv[c, pl.ds(g * 16, 16)]
                    for r in range(16):
                        w = w16[r]
                        for j in range(4):
                            sl = pl.ds(j * 16, 16)
                            rows[g * 16 + r, sl] = rows[g * 16 + r, sl] * w

                pltpu.async_copy(rows, acc.at[dstv.at[c]], ssems[k], add=True)
                nb = (k + 2) % 4
                nrows = rbufs[nb]

                @pl.when(4 * qq + k + 2 < half)
                def _prefetch():
                    @pl.when(4 * qq + k - 2 >= 0)
                    def _drain():
                        pltpu.make_async_copy(
                            nrows, acc.at[dstv.at[c]], ssems[nb]).wait()

                    pltpu.make_async_copy(
                        hp_hbm.at[srcv.at[c + 2]], nrows, gsems[nb]).start()

        for k in range(4):
            pltpu.make_async_copy(
                rbufs[k], acc.at[dstv.at[0]], ssems[k]).wait()
        plsc.subcore_barrier()
        pltpu.sync_copy(acc.at[pl.ds(sid * spt, spt)],
                        out_hbm.at[pl.ds((cid * 2 + q) * N + sid * spt, spt)])


def _sc_gcn_agg(hp, srcg, dstl, ew, zeros_acc):
    kfn = pl.kernel(
        _sc_gcn_body,
        out_type=jax.ShapeDtypeStruct((NT, FP), _f32),
        mesh=_mesh(),
        compiler_params=_NOLAYOUT,
        scratch_types=[
            pltpu.VMEM((NCH // 2, CH), jnp.int32),
            pltpu.VMEM((NCH // 2, CH), jnp.int32),
            pltpu.VMEM((NCH // 2, CH), _f32),
            pltpu.VMEM((CH, FP), _f32),
            pltpu.VMEM((CH, FP), _f32),
            pltpu.VMEM((CH, FP), _f32),
            pltpu.VMEM((CH, FP), _f32),
            pltpu.VMEM_SHARED((N, FP), _f32),
            pltpu.SemaphoreType.DMA,
            pltpu.SemaphoreType.DMA,
            pltpu.SemaphoreType.DMA,
            pltpu.SemaphoreType.DMA,
            pltpu.SemaphoreType.DMA,
            pltpu.SemaphoreType.DMA,
            pltpu.SemaphoreType.DMA,
            pltpu.SemaphoreType.DMA,
        ],
    )
    return kfn(hp, srcg, dstl, ew, zeros_acc)


def _sc_gat_body(hg_hbm, src_hbm, dst_hbm, es_hbm, ed_hbm, z_hbm,
                 out_hbm, outden_hbm,
                 srcv, dstv, esv, edv, rows0, rows1, wbuf, dtab,
                 acc, gsem0, gsem1, *, nh, f):
    dcols = 128 * nh
    dshift = 7 + {1: 0, 2: 1, 4: 2}[nh]
    cid = lax.axis_index("c")
    sid = lax.axis_index("s")
    wid = cid * NS + sid
    pltpu.sync_copy(es_hbm, esv)
    pltpu.sync_copy(ed_hbm, edv)
    zero16 = jnp.zeros((16,), _f32)

    @pl.loop(0, 32)
    def _zero(i):
        for j in range(dcols // 16):
            dtab[i, pl.ds(j * 16, 16)] = zero16

    rows_per_tile = N // NS
    pltpu.sync_copy(z_hbm.at[pl.ds(sid * rows_per_tile, rows_per_tile)],
                    acc.at[pl.ds(sid * rows_per_tile, rows_per_tile)])
    plsc.subcore_barrier()

    rbufs = (rows0, rows1)
    gsems = (gsem0, gsem1)
    half = NCH // 2
    for p in range(2):
        pltpu.sync_copy(src_hbm.at[wid].at[pl.ds(p * half, half)], srcv)
        pltpu.sync_copy(dst_hbm.at[wid].at[pl.ds(p * half, half)], dstv)
        for b in range(2):
            pltpu.make_async_copy(
                hg_hbm.at[srcv.at[b]], rbufs[b], gsems[b]).start()

        @pl.loop(0, half // 2)
        def _pair(pp):
            for k in range(2):
                c = 2 * pp + k
                rows = rbufs[k]

                @pl.loop(0, CH // 16)
                def _att(g):
                    s = srcv[c, pl.ds(g * 16, 16)]
                    d = dstv[c, pl.ds(g * 16, 16)]
                    for h in range(nh):
                        es_g = plsc.load_gather(esv, [s * nh + h])
                        ed_g = plsc.load_gather(edv, [d * nh + h])
                        w = _leaky_exp(es_g + ed_g)
                        wbuf[h, pl.ds(g * 16, 16)] = w
                        di = d * nh + h
                        row = lax.shift_right_logical(di, dshift)
                        col = lax.bitwise_and(di, dcols - 1)
                        plsc.addupdate_scatter(dtab, [row, col], w)

                pltpu.make_async_copy(
                    hg_hbm.at[srcv.at[c]], rows, gsems[k]).wait()

                @pl.loop(0, CH // 16)
                def _scale(g):
                    for h in range(nh):
                        w16 = wbuf[h, pl.ds(g * 16, 16)]
                        for r in range(16):
                            w = w16[r]
                            for j in range(4):
                                sl = pl.ds(h * 64 + j * 16, 16)
                                rows[g * 16 + r, sl] = rows[g * 16 + r, sl] * w

                pltpu.sync_copy(rows, acc.at[dstv.at[c]], add=True)

                @pl.when(2 * pp + k + 2 < half)
                def _prefetch():
                    pltpu.make_async_copy(
                        hg_hbm.at[srcv.at[c + 2]], rows, gsems[k]).start()

    plsc.subcore_barrier()
    pltpu.sync_copy(acc.at[pl.ds(sid * rows_per_tile, rows_per_tile)],
                    out_hbm.at[cid].at[pl.ds(sid * rows_per_tile, rows_per_tile)])
    pltpu.sync_copy(dtab, outden_hbm.at[wid])


def _sc_gat_agg(hg, src, dst, es_flat, ed_flat, zeros_acc, nh):
    f = hg.shape[1]
    kfn = pl.kernel(
        functools.partial(_sc_gat_body, nh=nh, f=f),
        out_type=(
            jax.ShapeDtypeStruct((NC, N, f), _f32),
            jax.ShapeDtypeStruct((NW, 32, 128 * nh), _f32),
        ),
        mesh=_mesh(),
        compiler_params=_NOLAYOUT,
        scratch_types=[
            pltpu.VMEM((NCH // 2, CH), jnp.int32),
            pltpu.VMEM((NCH // 2, CH), jnp.int32),
            pltpu.VMEM((N * nh,), _f32),
            pltpu.VMEM((N * nh,), _f32),
            pltpu.VMEM((CH, f), _f32),
            pltpu.VMEM((CH, f), _f32),
            pltpu.VMEM((4, CH), _f32),
            pltpu.VMEM((32, 128 * nh), _f32),
            pltpu.VMEM_SHARED((N, f), _f32),
            pltpu.SemaphoreType.DMA,
            pltpu.SemaphoreType.DMA,
        ],
    )
    return kfn(hg, src, dst, es_flat, ed_flat, zeros_acc)


def _sc_link_body(z_hbm, a_hbm, b_hbm, out_hbm, av, bv, ra, rb, ob):
    cid = lax.axis_index("c")
    sid = lax.axis_index("s")
    wid = cid * NS + sid
    pltpu.sync_copy(a_hbm.at[wid], av)
    pltpu.sync_copy(b_hbm.at[wid], bv)
    lanes = lax.iota(jnp.int32, 16)

    @pl.loop(0, LCH)
    def _chunk(c):
        pltpu.sync_copy(z_hbm.at[av.at[c]], ra)
        pltpu.sync_copy(z_hbm.at[bv.at[c]], rb)

        @pl.loop(0, CH // 16)
        def _group(g):
            out = jnp.zeros((16,), _f32)
            for r in range(16):
                rr = g * 16 + r
                acc = ra[rr, pl.ds(0, 16)] * rb[rr, pl.ds(0, 16)]
                for j in range(1, 4):
                    acc = acc + ra[rr, pl.ds(j * 16, 16)] * rb[rr, pl.ds(j * 16, 16)]
                out = jnp.where(lanes == r, jnp.sum(acc), out)
            ob[pl.ds(g * 16, 16)] = out

        pltpu.sync_copy(ob, out_hbm.at[pl.ds(wid * LPW + c * CH, CH)])


def _sc_link(z, aidx, bidx):
    kfn = pl.kernel(
        _sc_link_body,
        out_type=jax.ShapeDtypeStruct((E,), _f32),
        mesh=_mesh(),
        compiler_params=_NOLAYOUT,
        scratch_types=[
            pltpu.VMEM((LCH, CH), jnp.int32),
            pltpu.VMEM((LCH, CH), jnp.int32),
            pltpu.VMEM((CH, FP), _f32),
            pltpu.VMEM((CH, FP), _f32),
            pltpu.VMEM((CH,), _f32),
        ],
    )
    return kfn(z, aidx, bidx)


def kernel(x, edge_index, edge_attr, edge_label_index, W_gat1, a_src1, a_dst1,
           b_gat1, W_gat2, a_src2, a_dst2, b_gat2, W_c1, b_c1, W_c2, b_c2,
           W_hop, b_hop):
    ei = edge_index.astype(jnp.int32)
    eli = edge_label_index.astype(jnp.int32)
    x2d = x.reshape(NT, D_IN)

    offs = (jnp.arange(SEQ, dtype=jnp.int32) * N)[:, None]
    src_raw = ei[:, 0, :].reshape(NW, NCH, CH)
    dst_raw = ei[:, 1, :].reshape(NW, NCH, CH)
    src_g_flat = ei[:, 0, :] + offs
    dst_g = (ei[:, 1, :] + offs).reshape(NW, NCH, CH)

    def _gcn_layout(a):
        b = a.reshape(2, 2, NS, NCH // 2, CH)
        return b.transpose(0, 2, 1, 3, 4).reshape(NW, NCH, CH)

    src_gcn = _gcn_layout(src_g_flat)
    dst_gcn = _gcn_layout(ei[:, 1, :])
    ew_gcn = _gcn_layout(edge_attr)
    ew = edge_attr.reshape(NW, NCH, CH)
    aidx = eli[0].reshape(NW, LCH, CH)
    bidx = eli[1].reshape(NW, LCH, CH)

    W_hop_f = W_hop[:HID] + W_hop[HID:]
    W_c1p = jnp.concatenate([W_c1, jnp.zeros((D_IN, FP - HID), _f32)], axis=1)
    W_c2p = jnp.concatenate([W_c2, jnp.zeros((HID, FP - HID), _f32)], axis=1)
    W_g2p = jnp.concatenate(
        [W_gat2, jnp.zeros((H1 * HID, FP - OUT), _f32)], axis=1)
    b1 = b_c1.reshape(1, HID)
    b2 = b_c2.reshape(1, HID)
    bh = b_hop.reshape(1, N)
    bg1 = b_gat1.reshape(1, H1 * HID)
    bg2 = jnp.concatenate([b_gat2, jnp.zeros((FP - OUT,), _f32)]).reshape(1, FP)
    eye4 = jnp.eye(H1, dtype=_f32)
    A1 = jnp.concatenate(
        [(eye4[:, None, :] * a_src1[:, :, None]).reshape(H1 * HID, H1),
         (eye4[:, None, :] * a_dst1[:, :, None]).reshape(H1 * HID, H1)], axis=1)
    A2 = jnp.concatenate(
        [a_src2.T, a_dst2.T, jnp.zeros((OUT, 6), _f32)], axis=1)
    A2p = jnp.concatenate([A2, jnp.zeros((FP - OUT, 8), _f32)], axis=0)
    R4 = jnp.repeat(eye4, HID, axis=1)
    R1 = jnp.concatenate(
        [jnp.ones((1, OUT), _f32), jnp.zeros((1, FP - OUT), _f32)], axis=1)

    zeros_acc = jnp.zeros((N, FP), _f32)

    degp = _sc_deg(dst_g, ew).reshape(NC, NT, 1)

    bm = 256
    hp1, dinv = pl.pallas_call(
        _t1_body,
        grid=(NT // bm,),
        in_specs=[
            pl.BlockSpec((bm, D_IN), lambda i: (i, 0)),
            pl.BlockSpec((D_IN, FP), lambda i: (0, 0)),
            pl.BlockSpec((NC, bm, 1), lambda i: (0, i, 0)),
        ],
        out_specs=[
            pl.BlockSpec((bm, FP), lambda i: (i, 0)),
            pl.BlockSpec((bm, 1), lambda i: (i, 0)),
        ],
        out_shape=[
            jax.ShapeDtypeStruct((NT, FP), _f32),
            jax.ShapeDtypeStruct((NT, 1), _f32),
        ],
    )(x2d, W_c1p, degp)

    agg1 = _sc_gcn_agg(hp1, src_gcn, dst_gcn, ew_gcn, zeros_acc)

    hp2 = pl.pallas_call(
        _t2_body,
        grid=(NT // bm,),
        in_specs=[
            pl.BlockSpec((bm, FP), lambda i: (i, 0)),
            pl.BlockSpec((bm, FP), lambda i: (i, 0)),
            pl.BlockSpec((bm, 1), lambda i: (i, 0)),
            pl.BlockSpec((1, HID), lambda i: (0, 0)),
            pl.BlockSpec((HID, FP), lambda i: (0, 0)),
        ],
        out_specs=pl.BlockSpec((bm, FP), lambda i: (i, 0)),
        out_shape=jax.ShapeDtypeStruct((NT, FP), _f32),
    )(agg1, hp1, dinv, b1, W_c2p)

    agg2 = _sc_gcn_agg(hp2, src_gcn, dst_gcn, ew_gcn, zeros_acc)

    hg1, esd1 = _mm_epilogue_call(x2d, W_gat1, A1, bm)
    pn_halves, pd_halves = [], []
    for q in range(2):
        es_q = esd1[:N, 2 * q:2 * q + 2].reshape(-1)
        ed_q = esd1[:N, H1 + 2 * q:H1 + 2 * q + 2].reshape(-1)
        pn_q, pd_q = _sc_gat_agg(hg1[:, 128 * q:128 * (q + 1)], src_raw,
                                 dst_raw, es_q, ed_q, zeros_acc, 2)
        pn_halves.append(pn_q)
        pd_halves.append(pd_q.reshape(NW, N, 2))
    pn1 = jnp.concatenate(pn_halves, axis=2)
    pd1r = jnp.concatenate(pd_halves, axis=2)

    z1_low = pl.pallas_call(
        functools.partial(_t5a_body, nh=H1, relu=True),
        grid=(N // bm,),
        in_specs=[
            pl.BlockSpec((NC, bm, H1 * HID), lambda i: (0, i, 0)),
            pl.BlockSpec((NW, bm, H1), lambda i: (0, i, 0)),
            pl.BlockSpec((bm, H1 * HID), lambda i: (i, 0)),
            pl.BlockSpec((bm, 8), lambda i: (i, 0)),
            pl.BlockSpec((1, H1 * HID), lambda i: (0, 0)),
            pl.BlockSpec((H1, H1 * HID), lambda i: (0, 0)),
        ],
        out_specs=pl.BlockSpec((bm, H1 * HID), lambda i: (i, 0)),
        out_shape=jax.ShapeDtypeStruct((N, H1 * HID), _f32),
    )(pn1, pd1r, hg1, esd1, bg1, R4)

    bhigh = 512
    z1_high = pl.pallas_call(
        functools.partial(_t5b_body, relu=True),
        grid=((NT - N) // bhigh,),
        in_specs=[
            pl.BlockSpec((bhigh, H1 * HID), lambda i: (i + N // bhigh, 0)),
            pl.BlockSpec((1, H1 * HID), lambda i: (0, 0)),
        ],
        out_specs=pl.BlockSpec((bhigh, H1 * HID), lambda i: (i, 0)),
        out_shape=jax.ShapeDtypeStruct((NT - N, H1 * HID), _f32),
    )(hg1, bg1)
    z1 = jnp.concatenate([z1_low, z1_high], axis=0)

    hg2, esd2 = _mm_epilogue_call(z1, W_g2p, A2p, bm)
    es2_flat = esd2[:N, 0]
    ed2_flat = esd2[:N, 1]
    pn2, pd2 = _sc_gat_agg(hg2, src_raw, dst_raw, es2_flat, ed2_flat,
                           zeros_acc, 1)
    pd2r = pd2.reshape(NW, N, 1)

    z2_low = pl.pallas_call(
        functools.partial(_t5a_body, nh=1, relu=False),
        grid=(N // bm,),
        in_specs=[
            pl.BlockSpec((NC, bm, FP), lambda i: (0, i, 0)),
            pl.BlockSpec((NW, bm, 1), lambda i: (0, i, 0)),
            pl.BlockSpec((bm, FP), lambda i: (i, 0)),
            pl.BlockSpec((bm, 8), lambda i: (i, 0)),
            pl.BlockSpec((1, FP), lambda i: (0, 0)),
            pl.BlockSpec((1, FP), lambda i: (0, 0)),
        ],
        out_specs=pl.BlockSpec((bm, FP), lambda i: (i, 0)),
        out_shape=jax.ShapeDtypeStruct((N, FP), _f32),
    )(pn2, pd2r, hg2, esd2, bg2, R1)

    z2_high = pl.pallas_call(
        functools.partial(_t5b_body, relu=False),
        grid=((NT - N) // bhigh,),
        in_specs=[
            pl.BlockSpec((bhigh, FP), lambda i: (i + N // bhigh, 0)),
            pl.BlockSpec((1, FP), lambda i: (0, 0)),
        ],
        out_specs=pl.BlockSpec((bhigh, FP), lambda i: (i, 0)),
        out_shape=jax.ShapeDtypeStruct((NT - N, FP), _f32),
    )(hg2, bg2)
    z2 = jnp.concatenate([z2_low, z2_high], axis=0)

    link_pred = _sc_link(z2, aidx, bidx)

    bmh = 512
    hop2d = pl.pallas_call(
        _t3_body,
        grid=(NT // bmh,),
        in_specs=[
            pl.BlockSpec((bmh, FP), lambda i: (i, 0)),
            pl.BlockSpec((bmh, FP), lambda i: (i, 0)),
            pl.BlockSpec((bmh, 1), lambda i: (i, 0)),
            pl.BlockSpec((1, HID), lambda i: (0, 0)),
            pl.BlockSpec((HID, N), lambda i: (0, 0)),
            pl.BlockSpec((1, N), lambda i: (0, 0)),
        ],
        out_specs=pl.BlockSpec((bmh, N), lambda i: (i, 0)),
        out_shape=jax.ShapeDtypeStruct((NT, N), _f32),
    )(agg2, hp2, dinv, b2, W_hop_f, bh)
    hop_out = hop2d.reshape(SEQ, N, N)
    return (link_pred, hop_out)
